# Initial kernel scaffold; baseline (speedup 1.0000x reference)
import jax
import jax.numpy as jnp
from jax.experimental import pallas as pl


def kernel(x, edge_index, batch, emb, W1, b1, gamma1, beta1, W2, b2, Ws, bs):
    raise NotImplementedError("write your pallas kernel here")


# speedup vs baseline: 26.4621x; 26.4621x over previous
import functools

import jax
import jax.numpy as jnp
from jax import lax
from jax.experimental import pallas as pl
from jax.experimental.pallas import tpu as pltpu
from jax.experimental.pallas import tpu_sc as plsc

N = 10000
E = 160000
V = 64
D = 256
H = 512
K = 16
B = 100
NPER = 100

NC = 2
NS = 16
NW = NC * NS

EPT = 5120
EPAD = EPT * NW
ECH = 128
NECH = EPT // ECH

NPT = 384
NPAD = NPT * NW
NNCH = NPT // ECH

CNT_DUMP = N * V
CNT_SZ = 640128
CNT_SL = CNT_SZ // NS
Q_DUMP = N * NPER
Q_SZ = 1000064
Q_SL = Q_SZ // NS

STG = 8192


def _chunks(total):
    offs = []
    o = 0
    while o < total:
        offs.append((o, min(STG, total - o)))
        o += STG
    return offs


def _sc_body(src_hbm, dst_hbm, x0w_hbm, cnt_out, q_out,
             x0w_v, src_v, dst_v, cbuf, qbuf, ones_v, stage_v, cnt_sh, q_sh):
    c = lax.axis_index("c")
    s = lax.axis_index("s")
    wid = s * NC + c

    def zfill(i, _):
        stage_v[pl.ds(i * 16, 16)] = jnp.zeros((16,), jnp.float32)
        return ()

    lax.fori_loop(0, STG // 16, zfill, ())
    for off, ln in _chunks(CNT_SL):
        pltpu.sync_copy(stage_v.at[pl.ds(0, ln)],
                        cnt_sh.at[pl.ds(s * CNT_SL + off, ln)])
    for off, ln in _chunks(Q_SL):
        pltpu.sync_copy(stage_v.at[pl.ds(0, ln)],
                        q_sh.at[pl.ds(s * Q_SL + off, ln)])

    pltpu.sync_copy(x0w_hbm, x0w_v)
    pltpu.sync_copy(src_hbm.at[pl.ds(wid * EPT, EPT)], src_v)
    pltpu.sync_copy(dst_hbm.at[pl.ds(wid * EPT, EPT)], dst_v)

    for g in range(ECH // 16):
        ones_v[pl.ds(g * 16, 16)] = jnp.ones((16,), jnp.float32)

    plsc.subcore_barrier()

    def lookup_x0(iv):
        wv = plsc.load_gather(x0w_v, [lax.shift_right_logical(iv, 2)])
        sh = (iv & 3) * 8
        return lax.shift_right_logical(wv, sh) & 63

    def edge_chunk(ci, _):
        base = ci * ECH
        for g in range(ECH // 16):
            o = base + g * 16
            sv = src_v[pl.ds(o, 16)]
            dv = dst_v[pl.ds(o, 16)]
            xv = lookup_x0(sv)
            cbuf[pl.ds(g * 16, 16)] = jnp.minimum(dv * V + xv, CNT_DUMP)
            qbuf[pl.ds(g * 16, 16)] = jnp.minimum(
                sv * NPER + (dv % NPER), Q_DUMP)
        pltpu.sync_copy(ones_v, cnt_sh.at[cbuf], add=True)
        pltpu.sync_copy(ones_v, q_sh.at[qbuf], add=True)
        return ()

    lax.fori_loop(0, NECH, edge_chunk, ())

    def node_chunk(ci, _):
        base = wid * NPT + ci * ECH
        for g in range(ECH // 16):
            nv = base + g * 16 + lax.iota(jnp.int32, 16)
            xv = lookup_x0(nv)
            cbuf[pl.ds(g * 16, 16)] = jnp.minimum(nv * V + xv, CNT_DUMP)
        pltpu.sync_copy(ones_v, cnt_sh.at[cbuf], add=True)
        return ()

    lax.fori_loop(0, NNCH, node_chunk, ())

    plsc.subcore_barrier()

    for off, ln in _chunks(CNT_SL):
        pltpu.sync_copy(cnt_sh.at[pl.ds(s * CNT_SL + off, ln)],
                        stage_v.at[pl.ds(0, ln)])
        pltpu.sync_copy(
            stage_v.at[pl.ds(0, ln)],
            cnt_out.at[pl.ds(c * CNT_SZ + s * CNT_SL + off, ln)])
    for off, ln in _chunks(Q_SL):
        pltpu.sync_copy(q_sh.at[pl.ds(s * Q_SL + off, ln)],
                        stage_v.at[pl.ds(0, ln)])
        pltpu.sync_copy(
            stage_v.at[pl.ds(0, ln)],
            q_out.at[pl.ds(c * Q_SZ + s * Q_SL + off, ln)])


@functools.cache
def _sc_scatter_fn():
    return pl.kernel(
        _sc_body,
        out_type=(
            jax.ShapeDtypeStruct((NC * CNT_SZ,), jnp.float32),
            jax.ShapeDtypeStruct((NC * Q_SZ,), jnp.float32),
        ),
        mesh=plsc.VectorSubcoreMesh(
            core_axis_name="c", subcore_axis_name="s",
            num_cores=NC, num_subcores=NS),
        compiler_params=pltpu.CompilerParams(needs_layout_passes=False),
        scratch_types=[
            pltpu.VMEM((NPAD // 4,), jnp.int32),
            pltpu.VMEM((EPT,), jnp.int32),
            pltpu.VMEM((EPT,), jnp.int32),
            pltpu.VMEM((ECH,), jnp.int32),
            pltpu.VMEM((ECH,), jnp.int32),
            pltpu.VMEM((ECH,), jnp.float32),
            pltpu.VMEM((STG,), jnp.float32),
            pltpu.VMEM_SHARED((CNT_SZ,), jnp.float32),
            pltpu.VMEM_SHARED((Q_SZ,), jnp.float32),
        ],
    )


RB = 1000
NRB = N // RB


def _t1_body(cnt_ref, emb_ref, w1_ref, b1_ref, t_ref, st_ref, p1_scr, acc):
    i = pl.program_id(0)

    @pl.when(i == 0)
    def _():
        p1_scr[...] = jnp.dot(emb_ref[...], w1_ref[...],
                              preferred_element_type=jnp.float32)
        acc[...] = jnp.zeros_like(acc)

    m = cnt_ref[0] + cnt_ref[1]
    t_blk = jnp.dot(m, p1_scr[...],
                    preferred_element_type=jnp.float32) + b1_ref[...]
    t_ref[...] = t_blk
    acc[0:1, :] += jnp.sum(t_blk, axis=0, keepdims=True)
    acc[1:2, :] += jnp.sum(t_blk * t_blk, axis=0, keepdims=True)
    st_ref[...] = acc[...]


_t1_in_specs = [
    pl.BlockSpec((NC, RB, V), lambda i: (0, i, 0)),
    pl.BlockSpec((V, D), lambda i: (0, 0)),
    pl.BlockSpec((D, H), lambda i: (0, 0)),
    pl.BlockSpec((1, H), lambda i: (0, 0)),
]
_t1_out_specs = [
    pl.BlockSpec((RB, H), lambda i: (i, 0)),
    pl.BlockSpec((2, H), lambda i: (0, 0)),
]
_t1_out_shape = [
    jax.ShapeDtypeStruct((N, H), jnp.float32),
    jax.ShapeDtypeStruct((2, H), jnp.float32),
]
_t1_scratch = [
    pltpu.VMEM((V, H), jnp.float32),
    pltpu.VMEM((2, H), jnp.float32),
]

_t1 = pl.pallas_call(
    _t1_body,
    grid=(NRB,),
    in_specs=_t1_in_specs,
    out_specs=_t1_out_specs,
    out_shape=_t1_out_shape,
    scratch_shapes=_t1_scratch,
)

GPB = RB // NPER


def _t2_body(t_ref, st_ref, g1_ref, be_ref, w2_ref, b2_ref, ws_ref, bs_ref,
             q_ref, out_ref, acc_sm):
    i = pl.program_id(0)

    @pl.when(i == 0)
    def _():
        acc_sm[0] = 0.0

    mu = st_ref[0:1, :] * (1.0 / N)
    var = st_ref[1:2, :] * (1.0 / N) - mu * mu
    inv = lax.rsqrt(var + 1e-5)
    y = (t_ref[...] - mu) * inv * g1_ref[...] + be_ref[...]
    y = jnp.where(y > 0, y, jnp.exp(y) - 1.0)
    h2 = jnp.dot(y, w2_ref[...],
                 preferred_element_type=jnp.float32) + b2_ref[...]
    h2 = jnp.where(h2 > 0, h2, jnp.exp(h2) - 1.0)
    z = jnp.dot(h2, ws_ref[...],
                preferred_element_type=jnp.float32) + bs_ref[...]
    z = z - jnp.max(z, axis=-1, keepdims=True)
    ez = jnp.exp(z)
    sm = ez / jnp.sum(ez, axis=-1, keepdims=True)
    qs = q_ref[0] + q_ref[1]

    eye = (lax.broadcasted_iota(jnp.int32, (K, K), 0) ==
           lax.broadcasted_iota(jnp.int32, (K, K), 1)).astype(jnp.float32)

    tot = 0.0
    for g in range(GPB):
        sg = sm[g * NPER:(g + 1) * NPER, :]
        qb = qs[g * NPER:(g + 1) * NPER, :]
        deg = jnp.sum(qb, axis=1, keepdims=True)
        m2 = jnp.sum(deg)
        u = jnp.dot(qb, sg, preferred_element_type=jnp.float32)
        tr_out = jnp.sum(u * sg)
        cvec = jnp.sum(deg * sg, axis=0)
        tr_norm = jnp.sum(cvec * cvec) / m2
        spec = -(tr_out - tr_norm) / m2
        ss = lax.dot_general(sg, sg, (((0,), (0,)), ((), ())),
                             preferred_element_type=jnp.float32)
        fr = jnp.sqrt(jnp.sum(ss * ss))
        ortho = jnp.sqrt(jnp.sum((ss / fr - eye * 0.25) ** 2))
        cs = jnp.sum(sg, axis=0)
        clus = jnp.sqrt(jnp.sum(cs * cs)) * (4.0 / NPER) - 1.0
        tot = tot + spec + ortho + clus

    acc_sm[0] += tot
    out_ref[...] = jnp.broadcast_to(acc_sm[0] * (1.0 / B), (1, 1))


_t2_in_specs = [
    pl.BlockSpec((RB, H), lambda i: (i, 0)),
    pl.BlockSpec((2, H), lambda i: (0, 0)),
    pl.BlockSpec((1, H), lambda i: (0, 0)),
    pl.BlockSpec((1, H), lambda i: (0, 0)),
    pl.BlockSpec((H, H), lambda i: (0, 0)),
    pl.BlockSpec((1, H), lambda i: (0, 0)),
    pl.BlockSpec((H, K), lambda i: (0, 0)),
    pl.BlockSpec((1, K), lambda i: (0, 0)),
    pl.BlockSpec((NC, RB, NPER), lambda i: (0, i, 0)),
]
_t2_out_specs = pl.BlockSpec((1, 1), lambda i: (0, 0))
_t2_out_shape = jax.ShapeDtypeStruct((1, 1), jnp.float32)
_t2_scratch = [pltpu.SMEM((1,), jnp.float32)]

_t2 = pl.pallas_call(
    _t2_body,
    grid=(NRB,),
    in_specs=_t2_in_specs,
    out_specs=_t2_out_specs,
    out_shape=_t2_out_shape,
    scratch_shapes=_t2_scratch,
)


def kernel(x, edge_index, batch, emb, W1, b1, gamma1, beta1, W2, b2, Ws, bs):
    x0 = x[:, 0].astype(jnp.int32)
    x0p = jnp.concatenate([x0, jnp.zeros((NPAD - N,), jnp.int32)])
    x0q = x0p.reshape(NPAD // 4, 4)
    x0w = (x0q[:, 0] | (x0q[:, 1] << 8) | (x0q[:, 2] << 16)
           | (x0q[:, 3] << 24))
    src = edge_index[0].astype(jnp.int32)
    dst = edge_index[1].astype(jnp.int32)
    epad = jnp.full((EPAD - E,), N, jnp.int32)
    srcp = jnp.concatenate([src, epad])
    dstp = jnp.concatenate([dst, epad])

    cnt_flat, q_flat = _sc_scatter_fn()(srcp, dstp, x0w)
    cnt3 = cnt_flat.reshape(NC, CNT_SZ // V, V)[:, :N, :]
    q3 = q_flat.reshape(NC, Q_SZ)[:, :N * NPER].reshape(NC, N, NPER)

    t, stats = _t1(cnt3, emb, W1, b1.reshape(1, H))
    out = _t2(t, stats, gamma1.reshape(1, H), beta1.reshape(1, H),
              W2, b2.reshape(1, H), Ws, bs.reshape(1, K), q3)
    return out.reshape(())


# speedup vs baseline: 27.6594x; 1.0452x over previous
import functools

import jax
import jax.numpy as jnp
from jax import lax
from jax.experimental import pallas as pl
from jax.experimental.pallas import tpu as pltpu
from jax.experimental.pallas import tpu_sc as plsc

N = 10000
E = 160000
V = 64
D = 256
H = 512
K = 16
B = 100
NPER = 100

NC = 2
NS = 16
NW = NC * NS

EPT = 5120
EPAD = EPT * NW
ECH = 128
NECH = EPT // ECH

NPT = 384
NPAD = NPT * NW
NNCH = NPT // ECH

CNT_DUMP = N * V
CNT_SZ = 640128
CNT_SL = CNT_SZ // NS
Q_DUMP = N * NPER
Q_SZ = 1001600
Q_SL = Q_SZ // NS

STG = 8192
NB = 4


def _chunks(total):
    offs = []
    o = 0
    while o < total:
        offs.append((o, min(STG, total - o)))
        o += STG
    return offs


def _sc_body(src_hbm, dst_hbm, x0w_hbm, cnt_out, q_out,
             x0w_v, src_v, dst_v, cbuf, qbuf, ones_v, stage_v, cnt_sh, q_sh,
             sem_c, sem_q):
    c = lax.axis_index("c")
    s = lax.axis_index("s")
    wid = s * NC + c

    def zfill(i, _):
        stage_v[pl.ds(i * 16, 16)] = jnp.zeros((16,), jnp.float32)
        return ()

    lax.fori_loop(0, STG // 16, zfill, ())
    for off, ln in _chunks(CNT_SL):
        pltpu.async_copy(stage_v.at[pl.ds(0, ln)],
                         cnt_sh.at[pl.ds(s * CNT_SL + off, ln)], sem_c)
    for off, ln in _chunks(Q_SL):
        pltpu.async_copy(stage_v.at[pl.ds(0, ln)],
                         q_sh.at[pl.ds(s * Q_SL + off, ln)], sem_q)

    pltpu.sync_copy(x0w_hbm, x0w_v)
    pltpu.sync_copy(src_hbm.at[pl.ds(wid * EPT, EPT)], src_v)
    pltpu.sync_copy(dst_hbm.at[pl.ds(wid * EPT, EPT)], dst_v)

    for g in range(ECH // 16):
        ones_v[pl.ds(g * 16, 16)] = jnp.ones((16,), jnp.float32)

    for off, ln in _chunks(CNT_SL):
        pltpu.make_async_copy(stage_v.at[pl.ds(0, ln)],
                              cnt_sh.at[pl.ds(s * CNT_SL + off, ln)],
                              sem_c).wait()
    for off, ln in _chunks(Q_SL):
        pltpu.make_async_copy(stage_v.at[pl.ds(0, ln)],
                              q_sh.at[pl.ds(s * Q_SL + off, ln)],
                              sem_q).wait()

    plsc.subcore_barrier()

    def lookup_x0(iv):
        wv = plsc.load_gather(x0w_v, [lax.shift_right_logical(iv, 2)])
        sh = (iv & 3) * 8
        return lax.shift_right_logical(wv, sh) & 63

    def edge_chunk(ci, b):
        base = ci * ECH
        for g in range(ECH // 16):
            o = base + g * 16
            sv = src_v[pl.ds(o, 16)]
            dv = dst_v[pl.ds(o, 16)]
            xv = lookup_x0(sv)
            cbuf[b, pl.ds(g * 16, 16)] = jnp.minimum(dv * V + xv, CNT_DUMP)
            qbuf[b, pl.ds(g * 16, 16)] = jnp.minimum(
                sv * NPER + (dv % NPER), Q_DUMP)

    def fire(b):
        pltpu.async_copy(ones_v, cnt_sh.at[cbuf.at[b]], sem_c, add=True)
        pltpu.async_copy(ones_v, q_sh.at[qbuf.at[b]], sem_q, add=True)

    def drain():
        pltpu.make_async_copy(ones_v, cnt_sh.at[cbuf.at[0]], sem_c).wait()
        pltpu.make_async_copy(ones_v, q_sh.at[qbuf.at[0]], sem_q).wait()

    for b in range(NB):
        edge_chunk(b, b)
        fire(b)

    def ring_body(g4, _):
        for b in range(NB):
            drain()
            edge_chunk(NB + g4 * NB + b, b)
            fire(b)
        return ()

    lax.fori_loop(0, (NECH - NB) // NB, ring_body, ())
    for b in range(NB):
        drain()

    def node_chunk(ci, _):
        base = wid * NPT + ci * ECH
        for g in range(ECH // 16):
            nv = base + g * 16 + lax.iota(jnp.int32, 16)
            xv = lookup_x0(nv)
            cbuf[0, pl.ds(g * 16, 16)] = jnp.minimum(nv * V + xv, CNT_DUMP)
        pltpu.sync_copy(ones_v, cnt_sh.at[cbuf.at[0]], add=True)
        return ()

    lax.fori_loop(0, NNCH, node_chunk, ())

    plsc.subcore_barrier()

    for off, ln in _chunks(CNT_SL):
        pltpu.sync_copy(cnt_sh.at[pl.ds(s * CNT_SL + off, ln)],
                        stage_v.at[pl.ds(0, ln)])
        pltpu.sync_copy(
            stage_v.at[pl.ds(0, ln)],
            cnt_out.at[pl.ds(c * CNT_SZ + s * CNT_SL + off, ln)])
    for off, ln in _chunks(Q_SL):
        pltpu.sync_copy(q_sh.at[pl.ds(s * Q_SL + off, ln)],
                        stage_v.at[pl.ds(0, ln)])
        pltpu.sync_copy(
            stage_v.at[pl.ds(0, ln)],
            q_out.at[pl.ds(c * Q_SZ + s * Q_SL + off, ln)])


@functools.cache
def _sc_scatter_fn():
    return pl.kernel(
        _sc_body,
        out_type=(
            jax.ShapeDtypeStruct((NC * CNT_SZ,), jnp.float32),
            jax.ShapeDtypeStruct((NC * Q_SZ,), jnp.float32),
        ),
        mesh=plsc.VectorSubcoreMesh(
            core_axis_name="c", subcore_axis_name="s",
            num_cores=NC, num_subcores=NS),
        compiler_params=pltpu.CompilerParams(needs_layout_passes=False),
        scratch_types=[
            pltpu.VMEM((NPAD // 4,), jnp.int32),
            pltpu.VMEM((EPT,), jnp.int32),
            pltpu.VMEM((EPT,), jnp.int32),
            pltpu.VMEM((NB, ECH), jnp.int32),
            pltpu.VMEM((NB, ECH), jnp.int32),
            pltpu.VMEM((ECH,), jnp.float32),
            pltpu.VMEM((STG,), jnp.float32),
            pltpu.VMEM_SHARED((CNT_SZ,), jnp.float32),
            pltpu.VMEM_SHARED((Q_SZ,), jnp.float32),
            pltpu.SemaphoreType.DMA,
            pltpu.SemaphoreType.DMA,
        ],
    )


RB = 1000
NRB = N // RB


def _t1_body(cnt_ref, emb_ref, w1_ref, b1_ref, t_ref, st_ref, p1_scr, acc):
    i = pl.program_id(0)

    @pl.when(i == 0)
    def _():
        p1_scr[...] = jnp.dot(emb_ref[...], w1_ref[...],
                              preferred_element_type=jnp.float32)
        acc[...] = jnp.zeros_like(acc)

    m = cnt_ref[0] + cnt_ref[1]
    t_blk = jnp.dot(m, p1_scr[...],
                    preferred_element_type=jnp.float32) + b1_ref[...]
    t_ref[...] = t_blk
    acc[0:1, :] += jnp.sum(t_blk, axis=0, keepdims=True)
    acc[1:2, :] += jnp.sum(t_blk * t_blk, axis=0, keepdims=True)
    st_ref[...] = acc[...]


_t1_in_specs = [
    pl.BlockSpec((NC, RB, V), lambda i: (0, i, 0)),
    pl.BlockSpec((V, D), lambda i: (0, 0)),
    pl.BlockSpec((D, H), lambda i: (0, 0)),
    pl.BlockSpec((1, H), lambda i: (0, 0)),
]
_t1_out_specs = [
    pl.BlockSpec((RB, H), lambda i: (i, 0)),
    pl.BlockSpec((2, H), lambda i: (0, 0)),
]
_t1_out_shape = [
    jax.ShapeDtypeStruct((N, H), jnp.float32),
    jax.ShapeDtypeStruct((2, H), jnp.float32),
]
_t1_scratch = [
    pltpu.VMEM((V, H), jnp.float32),
    pltpu.VMEM((2, H), jnp.float32),
]

_t1 = pl.pallas_call(
    _t1_body,
    grid=(NRB,),
    in_specs=_t1_in_specs,
    out_specs=_t1_out_specs,
    out_shape=_t1_out_shape,
    scratch_shapes=_t1_scratch,
)

GPB = RB // NPER


def _t2_body(t_ref, st_ref, g1_ref, be_ref, w2_ref, b2_ref, ws_ref, bs_ref,
             q_ref, out_ref, acc_sm):
    i = pl.program_id(0)

    @pl.when(i == 0)
    def _():
        acc_sm[0] = 0.0

    mu = st_ref[0:1, :] * (1.0 / N)
    var = st_ref[1:2, :] * (1.0 / N) - mu * mu
    inv = lax.rsqrt(var + 1e-5)
    y = (t_ref[...] - mu) * inv * g1_ref[...] + be_ref[...]
    y = jnp.where(y > 0, y, jnp.exp(y) - 1.0)
    h2 = jnp.dot(y, w2_ref[...],
                 preferred_element_type=jnp.float32) + b2_ref[...]
    h2 = jnp.where(h2 > 0, h2, jnp.exp(h2) - 1.0)
    z = jnp.dot(h2, ws_ref[...],
                preferred_element_type=jnp.float32) + bs_ref[...]
    z = z - jnp.max(z, axis=-1, keepdims=True)
    ez = jnp.exp(z)
    sm = ez / jnp.sum(ez, axis=-1, keepdims=True)
    qs = q_ref[0] + q_ref[1]

    eye = (lax.broadcasted_iota(jnp.int32, (K, K), 0) ==
           lax.broadcasted_iota(jnp.int32, (K, K), 1)).astype(jnp.float32)

    tot = 0.0
    for g in range(GPB):
        sg = sm[g * NPER:(g + 1) * NPER, :]
        qb = qs[g * NPER:(g + 1) * NPER, :]
        deg = jnp.sum(qb, axis=1, keepdims=True)
        m2 = jnp.sum(deg)
        u = jnp.dot(qb, sg, preferred_element_type=jnp.float32)
        tr_out = jnp.sum(u * sg)
        cvec = jnp.sum(deg * sg, axis=0)
        tr_norm = jnp.sum(cvec * cvec) / m2
        spec = -(tr_out - tr_norm) / m2
        ss = lax.dot_general(sg, sg, (((0,), (0,)), ((), ())),
                             preferred_element_type=jnp.float32)
        fr = jnp.sqrt(jnp.sum(ss * ss))
        ortho = jnp.sqrt(jnp.sum((ss / fr - eye * 0.25) ** 2))
        cs = jnp.sum(sg, axis=0)
        clus = jnp.sqrt(jnp.sum(cs * cs)) * (4.0 / NPER) - 1.0
        tot = tot + spec + ortho + clus

    acc_sm[0] += tot
    out_ref[...] = jnp.broadcast_to(acc_sm[0] * (1.0 / B), (1, 1))


_t2_in_specs = [
    pl.BlockSpec((RB, H), lambda i: (i, 0)),
    pl.BlockSpec((2, H), lambda i: (0, 0)),
    pl.BlockSpec((1, H), lambda i: (0, 0)),
    pl.BlockSpec((1, H), lambda i: (0, 0)),
    pl.BlockSpec((H, H), lambda i: (0, 0)),
    pl.BlockSpec((1, H), lambda i: (0, 0)),
    pl.BlockSpec((H, K), lambda i: (0, 0)),
    pl.BlockSpec((1, K), lambda i: (0, 0)),
    pl.BlockSpec((NC, RB, NPER), lambda i: (0, i, 0)),
]
_t2_out_specs = pl.BlockSpec((1, 1), lambda i: (0, 0))
_t2_out_shape = jax.ShapeDtypeStruct((1, 1), jnp.float32)
_t2_scratch = [pltpu.SMEM((1,), jnp.float32)]

_t2 = pl.pallas_call(
    _t2_body,
    grid=(NRB,),
    in_specs=_t2_in_specs,
    out_specs=_t2_out_specs,
    out_shape=_t2_out_shape,
    scratch_shapes=_t2_scratch,
)


def kernel(x, edge_index, batch, emb, W1, b1, gamma1, beta1, W2, b2, Ws, bs):
    x0 = x[:, 0].astype(jnp.int32)
    x0p = jnp.concatenate([x0, jnp.zeros((NPAD - N,), jnp.int32)])
    x0q = x0p.reshape(NPAD // 4, 4)
    x0w = (x0q[:, 0] | (x0q[:, 1] << 8) | (x0q[:, 2] << 16)
           | (x0q[:, 3] << 24))
    src = edge_index[0].astype(jnp.int32)
    dst = edge_index[1].astype(jnp.int32)
    epad = jnp.full((EPAD - E,), N, jnp.int32)
    srcp = jnp.concatenate([src, epad])
    dstp = jnp.concatenate([dst, epad])

    cnt_flat, q_flat = _sc_scatter_fn()(srcp, dstp, x0w)
    cnt3 = cnt_flat.reshape(NC, CNT_SZ // V, V)
    q3 = q_flat.reshape(NC, Q_SZ // NPER, NPER)

    t, stats = _t1(cnt3, emb, W1, b1.reshape(1, H))
    out = _t2(t, stats, gamma1.reshape(1, H), beta1.reshape(1, H),
              W2, b2.reshape(1, H), Ws, bs.reshape(1, K), q3)
    return out.reshape(())


# speedup vs baseline: 28.5859x; 1.0335x over previous
import functools

import jax
import jax.numpy as jnp
from jax import lax
from jax.experimental import pallas as pl
from jax.experimental.pallas import tpu as pltpu
from jax.experimental.pallas import tpu_sc as plsc

N = 10000
E = 160000
V = 64
D = 256
H = 512
K = 16
B = 100
NPER = 100

NC = 2
NS = 16

ROWS = 10016
ACC_SZ = ROWS * 128
DUMP = N * 128

EPT = 10240
EPAD = EPT * NS
ECH = 128
NECH = EPT // ECH
NB = 4

NPT = 640
NPAD = NPT * NS
NNCH = NPT // ECH

SLC = ACC_SZ // NS
STG = 8192


def _chunks(total):
    offs = []
    o = 0
    while o < total:
        offs.append((o, min(STG, total - o)))
        o += STG
    return offs


def _sc_body(src_hbm, dst_hbm, x0w_hbm, cnt_out, q_out,
             x0w_v, src_v, dst_v, ibuf, ones_v, stage_v, acc_sh,
             sem_s, sem_z):
    c = lax.axis_index("c")
    s = lax.axis_index("s")

    def zfill(i, _):
        stage_v[pl.ds(i * 16, 16)] = jnp.zeros((16,), jnp.float32)
        return ()

    lax.fori_loop(0, STG // 16, zfill, ())
    for off, ln in _chunks(SLC):
        pltpu.async_copy(stage_v.at[pl.ds(0, ln)],
                         acc_sh.at[pl.ds(s * SLC + off, ln)], sem_z)

    pltpu.sync_copy(src_hbm.at[pl.ds(s * EPT, EPT)], src_v)
    pltpu.sync_copy(dst_hbm.at[pl.ds(s * EPT, EPT)], dst_v)

    @pl.when(c == 0)
    def _():
        pltpu.sync_copy(x0w_hbm, x0w_v)

    for g in range(ECH // 16):
        ones_v[pl.ds(g * 16, 16)] = jnp.ones((16,), jnp.float32)

    for off, ln in _chunks(SLC):
        pltpu.make_async_copy(stage_v.at[pl.ds(0, ln)],
                              acc_sh.at[pl.ds(s * SLC + off, ln)],
                              sem_z).wait()

    plsc.subcore_barrier()

    def lookup_x0(iv):
        wv = plsc.load_gather(x0w_v, [lax.shift_right_logical(iv, 2)])
        sh = (iv & 3) * 8
        return lax.shift_right_logical(wv, sh) & 63

    def chunk_cnt(ci, b):
        base = ci * ECH
        for g in range(ECH // 16):
            o = base + g * 16
            sv = src_v[pl.ds(o, 16)]
            dv = dst_v[pl.ds(o, 16)]
            xv = lookup_x0(sv)
            ibuf[b, pl.ds(g * 16, 16)] = jnp.minimum(dv * 128 + xv, DUMP)

    def chunk_q(ci, b):
        base = ci * ECH
        for g in range(ECH // 16):
            o = base + g * 16
            sv = src_v[pl.ds(o, 16)]
            dv = dst_v[pl.ds(o, 16)]
            ibuf[b, pl.ds(g * 16, 16)] = jnp.minimum(
                sv * 128 + (dv % NPER), DUMP)

    def fire(b):
        pltpu.async_copy(ones_v, acc_sh.at[ibuf.at[b]], sem_s, add=True)

    def drain():
        pltpu.make_async_copy(ones_v, acc_sh.at[ibuf.at[0]], sem_s).wait()

    def run_ring(chunk_fn, nch):
        for b in range(NB):
            chunk_fn(b, b)
            fire(b)

        def ring_body(g4, _):
            for b in range(NB):
                drain()
                chunk_fn(NB + g4 * NB + b, b)
                fire(b)
            return ()

        lax.fori_loop(0, (nch - NB) // NB, ring_body, ())
        for b in range(NB):
            drain()

    @pl.when(c == 0)
    def _():
        run_ring(chunk_cnt, NECH)

        def node_chunk(ci, _):
            base = s * NPT + ci * ECH
            for g in range(ECH // 16):
                nv = base + g * 16 + lax.iota(jnp.int32, 16)
                xv = lookup_x0(nv)
                ibuf[0, pl.ds(g * 16, 16)] = jnp.minimum(nv * 128 + xv, DUMP)
            pltpu.sync_copy(ones_v, acc_sh.at[ibuf.at[0]], add=True)
            return ()

        lax.fori_loop(0, NNCH, node_chunk, ())

    @pl.when(c == 1)
    def _():
        run_ring(chunk_q, NECH)

    plsc.subcore_barrier()

    @pl.when(c == 0)
    def _():
        for off, ln in _chunks(SLC):
            pltpu.sync_copy(acc_sh.at[pl.ds(s * SLC + off, ln)],
                            stage_v.at[pl.ds(0, ln)])
            pltpu.sync_copy(stage_v.at[pl.ds(0, ln)],
                            cnt_out.at[pl.ds(s * SLC + off, ln)])

    @pl.when(c == 1)
    def _():
        for off, ln in _chunks(SLC):
            pltpu.sync_copy(acc_sh.at[pl.ds(s * SLC + off, ln)],
                            stage_v.at[pl.ds(0, ln)])
            pltpu.sync_copy(stage_v.at[pl.ds(0, ln)],
                            q_out.at[pl.ds(s * SLC + off, ln)])


@functools.cache
def _sc_scatter_fn():
    return pl.kernel(
        _sc_body,
        out_type=(
            jax.ShapeDtypeStruct((ACC_SZ,), jnp.float32),
            jax.ShapeDtypeStruct((ACC_SZ,), jnp.float32),
        ),
        mesh=plsc.VectorSubcoreMesh(
            core_axis_name="c", subcore_axis_name="s",
            num_cores=NC, num_subcores=NS),
        compiler_params=pltpu.CompilerParams(needs_layout_passes=False),
        scratch_types=[
            pltpu.VMEM((NPAD // 4,), jnp.int32),
            pltpu.VMEM((EPT,), jnp.int32),
            pltpu.VMEM((EPT,), jnp.int32),
            pltpu.VMEM((NB, ECH), jnp.int32),
            pltpu.VMEM((ECH,), jnp.float32),
            pltpu.VMEM((STG,), jnp.float32),
            pltpu.VMEM_SHARED((ACC_SZ,), jnp.float32),
            pltpu.SemaphoreType.DMA,
            pltpu.SemaphoreType.DMA,
        ],
    )


RB = 1000
NRB = N // RB


def _t1_body(cnt_ref, emb_ref, w1_ref, b1_ref, t_ref, st_ref, p1_scr, acc):
    i = pl.program_id(0)

    @pl.when(i == 0)
    def _():
        p1_scr[...] = jnp.zeros_like(p1_scr)
        p1_scr[0:V, :] = jnp.dot(emb_ref[...], w1_ref[...],
                                 preferred_element_type=jnp.float32)
        acc[...] = jnp.zeros_like(acc)

    t_blk = jnp.dot(cnt_ref[...], p1_scr[...],
                    preferred_element_type=jnp.float32) + b1_ref[...]
    t_ref[...] = t_blk
    acc[0:1, :] += jnp.sum(t_blk, axis=0, keepdims=True)
    acc[1:2, :] += jnp.sum(t_blk * t_blk, axis=0, keepdims=True)
    st_ref[...] = acc[...]


_t1_in_specs = [
    pl.BlockSpec((RB, 128), lambda i: (i, 0)),
    pl.BlockSpec((V, D), lambda i: (0, 0)),
    pl.BlockSpec((D, H), lambda i: (0, 0)),
    pl.BlockSpec((1, H), lambda i: (0, 0)),
]
_t1_out_specs = [
    pl.BlockSpec((RB, H), lambda i: (i, 0)),
    pl.BlockSpec((2, H), lambda i: (0, 0)),
]
_t1_out_shape = [
    jax.ShapeDtypeStruct((N, H), jnp.float32),
    jax.ShapeDtypeStruct((2, H), jnp.float32),
]
_t1_scratch = [
    pltpu.VMEM((128, H), jnp.float32),
    pltpu.VMEM((2, H), jnp.float32),
]

_t1 = pl.pallas_call(
    _t1_body,
    grid=(NRB,),
    in_specs=_t1_in_specs,
    out_specs=_t1_out_specs,
    out_shape=_t1_out_shape,
    scratch_shapes=_t1_scratch,
)

GPB = RB // NPER


def _t2_body(t_ref, st_ref, g1_ref, be_ref, w2_ref, b2_ref, ws_ref, bs_ref,
             q_ref, out_ref, acc_sm):
    i = pl.program_id(0)

    @pl.when(i == 0)
    def _():
        acc_sm[0] = 0.0

    mu = st_ref[0:1, :] * (1.0 / N)
    var = st_ref[1:2, :] * (1.0 / N) - mu * mu
    inv = lax.rsqrt(var + 1e-5)
    y = (t_ref[...] - mu) * inv * g1_ref[...] + be_ref[...]
    y = jnp.where(y > 0, y, jnp.exp(y) - 1.0)
    h2 = jnp.dot(y, w2_ref[...],
                 preferred_element_type=jnp.float32) + b2_ref[...]
    h2 = jnp.where(h2 > 0, h2, jnp.exp(h2) - 1.0)
    z = jnp.dot(h2, ws_ref[...],
                preferred_element_type=jnp.float32) + bs_ref[...]
    z = z - jnp.max(z, axis=-1, keepdims=True)
    ez = jnp.exp(z)
    sm = ez / jnp.sum(ez, axis=-1, keepdims=True)

    eye = (lax.broadcasted_iota(jnp.int32, (K, K), 0) ==
           lax.broadcasted_iota(jnp.int32, (K, K), 1)).astype(jnp.float32)
    zpad = jnp.zeros((128 - NPER, K), jnp.float32)

    tot = 0.0
    for g in range(GPB):
        sg = sm[g * NPER:(g + 1) * NPER, :]
        qb = q_ref[g * NPER:(g + 1) * NPER, :]
        deg = jnp.sum(qb, axis=1, keepdims=True)
        m2 = jnp.sum(deg)
        spad = jnp.concatenate([sg, zpad], axis=0)
        u = jnp.dot(qb, spad, preferred_element_type=jnp.float32)
        tr_out = jnp.sum(u * sg)
        cvec = jnp.sum(deg * sg, axis=0)
        tr_norm = jnp.sum(cvec * cvec) / m2
        spec = -(tr_out - tr_norm) / m2
        ss = lax.dot_general(sg, sg, (((0,), (0,)), ((), ())),
                             preferred_element_type=jnp.float32)
        fr = jnp.sqrt(jnp.sum(ss * ss))
        ortho = jnp.sqrt(jnp.sum((ss / fr - eye * 0.25) ** 2))
        cs = jnp.sum(sg, axis=0)
        clus = jnp.sqrt(jnp.sum(cs * cs)) * (4.0 / NPER) - 1.0
        tot = tot + spec + ortho + clus

    acc_sm[0] += tot
    out_ref[...] = jnp.broadcast_to(acc_sm[0] * (1.0 / B), (1, 1))


_t2_in_specs = [
    pl.BlockSpec((RB, H), lambda i: (i, 0)),
    pl.BlockSpec((2, H), lambda i: (0, 0)),
    pl.BlockSpec((1, H), lambda i: (0, 0)),
    pl.BlockSpec((1, H), lambda i: (0, 0)),
    pl.BlockSpec((H, H), lambda i: (0, 0)),
    pl.BlockSpec((1, H), lambda i: (0, 0)),
    pl.BlockSpec((H, K), lambda i: (0, 0)),
    pl.BlockSpec((1, K), lambda i: (0, 0)),
    pl.BlockSpec((RB, 128), lambda i: (i, 0)),
]
_t2_out_specs = pl.BlockSpec((1, 1), lambda i: (0, 0))
_t2_out_shape = jax.ShapeDtypeStruct((1, 1), jnp.float32)
_t2_scratch = [pltpu.SMEM((1,), jnp.float32)]

_t2 = pl.pallas_call(
    _t2_body,
    grid=(NRB,),
    in_specs=_t2_in_specs,
    out_specs=_t2_out_specs,
    out_shape=_t2_out_shape,
    scratch_shapes=_t2_scratch,
)


def kernel(x, edge_index, batch, emb, W1, b1, gamma1, beta1, W2, b2, Ws, bs):
    x0 = x[:, 0].astype(jnp.int32)
    x0p = jnp.concatenate([x0, jnp.zeros((NPAD - N,), jnp.int32)])
    x0q = x0p.reshape(NPAD // 4, 4)
    x0w = (x0q[:, 0] | (x0q[:, 1] << 8) | (x0q[:, 2] << 16)
           | (x0q[:, 3] << 24))
    src = edge_index[0].astype(jnp.int32)
    dst = edge_index[1].astype(jnp.int32)
    epad = jnp.full((EPAD - E,), N, jnp.int32)
    srcp = jnp.concatenate([src, epad])
    dstp = jnp.concatenate([dst, epad])

    cnt_flat, q_flat = _sc_scatter_fn()(srcp, dstp, x0w)
    cnt2 = cnt_flat.reshape(ROWS, 128)
    q2 = q_flat.reshape(ROWS, 128)

    t, stats = _t1(cnt2, emb, W1, b1.reshape(1, H))
    out = _t2(t, stats, gamma1.reshape(1, H), beta1.reshape(1, H),
              W2, b2.reshape(1, H), Ws, bs.reshape(1, K), q2)
    return out.reshape(())


# speedup vs baseline: 30.0061x; 1.0497x over previous
import functools

import jax
import jax.numpy as jnp
from jax import lax
from jax.experimental import pallas as pl
from jax.experimental.pallas import tpu as pltpu
from jax.experimental.pallas import tpu_sc as plsc

N = 10000
E = 160000
V = 64
D = 256
H = 512
K = 16
B = 100
NPER = 100

NC = 2
NS = 16
NW = NC * NS

ROWS = 10016
ACC_SZ = ROWS * 128
DUMP = N * 128

EPT = 5120
EPAD = EPT * NW
ECH = 128
NECH = EPT // ECH
NB = 4

NPT = 384
NPAD = NPT * NW
NNCH = NPT // ECH

SLC = ACC_SZ // NS
STG = 8192


def _chunks(total):
    offs = []
    o = 0
    while o < total:
        offs.append((o, min(STG, total - o)))
        o += STG
    return offs


def _zero_acc(s, stage_v, acc_sh, sem_z):
    def zfill(i, _):
        stage_v[pl.ds(i * 16, 16)] = jnp.zeros((16,), jnp.float32)
        return ()

    lax.fori_loop(0, STG // 16, zfill, ())
    for off, ln in _chunks(SLC):
        pltpu.async_copy(stage_v.at[pl.ds(0, ln)],
                         acc_sh.at[pl.ds(s * SLC + off, ln)], sem_z)


def _drain_zero(s, stage_v, acc_sh, sem_z):
    for off, ln in _chunks(SLC):
        pltpu.make_async_copy(stage_v.at[pl.ds(0, ln)],
                              acc_sh.at[pl.ds(s * SLC + off, ln)],
                              sem_z).wait()


def _copy_out(c, s, stage_v, acc_sh, out_hbm):
    for off, ln in _chunks(SLC):
        pltpu.sync_copy(acc_sh.at[pl.ds(s * SLC + off, ln)],
                        stage_v.at[pl.ds(0, ln)])
        pltpu.sync_copy(stage_v.at[pl.ds(0, ln)],
                        out_hbm.at[pl.ds(c * ACC_SZ + s * SLC + off, ln)])


def _run_ring(chunk_fn, nch, ibuf, ones_v, acc_sh, sem_s):
    def fire(b):
        pltpu.async_copy(ones_v, acc_sh.at[ibuf.at[b]], sem_s, add=True)

    def drain():
        pltpu.make_async_copy(ones_v, acc_sh.at[ibuf.at[0]], sem_s).wait()

    for b in range(NB):
        chunk_fn(b, b)
        fire(b)

    def ring_body(g4, _):
        for b in range(NB):
            drain()
            chunk_fn(NB + g4 * NB + b, b)
            fire(b)
        return ()

    lax.fori_loop(0, (nch - NB) // NB, ring_body, ())
    for b in range(NB):
        drain()


def _sc_cnt_body(src_hbm, dst_hbm, x0w_hbm, cnt_out,
                 x0w_v, src_v, dst_v, ibuf, ones_v, stage_v, acc_sh,
                 sem_s, sem_z):
    c = lax.axis_index("c")
    s = lax.axis_index("s")
    wid = s * NC + c

    _zero_acc(s, stage_v, acc_sh, sem_z)
    pltpu.sync_copy(src_hbm.at[pl.ds(wid * EPT, EPT)], src_v)
    pltpu.sync_copy(dst_hbm.at[pl.ds(wid * EPT, EPT)], dst_v)
    pltpu.sync_copy(x0w_hbm, x0w_v)
    for g in range(ECH // 16):
        ones_v[pl.ds(g * 16, 16)] = jnp.ones((16,), jnp.float32)
    _drain_zero(s, stage_v, acc_sh, sem_z)
    plsc.subcore_barrier()

    def lookup_x0(iv):
        wv = plsc.load_gather(x0w_v, [lax.shift_right_logical(iv, 2)])
        sh = (iv & 3) * 8
        return lax.shift_right_logical(wv, sh) & 63

    def chunk_cnt(ci, b):
        base = ci * ECH
        for g in range(ECH // 16):
            o = base + g * 16
            sv = src_v[pl.ds(o, 16)]
            dv = dst_v[pl.ds(o, 16)]
            xv = lookup_x0(sv)
            ibuf[b, pl.ds(g * 16, 16)] = jnp.minimum(dv * 128 + xv, DUMP)

    _run_ring(chunk_cnt, NECH, ibuf, ones_v, acc_sh, sem_s)

    def node_chunk(ci, _):
        base = wid * NPT + ci * ECH
        for g in range(ECH // 16):
            nv = base + g * 16 + lax.iota(jnp.int32, 16)
            xv = lookup_x0(nv)
            ibuf[0, pl.ds(g * 16, 16)] = jnp.minimum(nv * 128 + xv, DUMP)
        pltpu.sync_copy(ones_v, acc_sh.at[ibuf.at[0]], add=True)
        return ()

    lax.fori_loop(0, NNCH, node_chunk, ())
    plsc.subcore_barrier()
    _copy_out(c, s, stage_v, acc_sh, cnt_out)


def _sc_q_body(src_hbm, dst_hbm, q_out,
               src_v, dst_v, ibuf, ones_v, stage_v, acc_sh, sem_s, sem_z):
    c = lax.axis_index("c")
    s = lax.axis_index("s")
    wid = s * NC + c

    _zero_acc(s, stage_v, acc_sh, sem_z)
    pltpu.sync_copy(src_hbm.at[pl.ds(wid * EPT, EPT)], src_v)
    pltpu.sync_copy(dst_hbm.at[pl.ds(wid * EPT, EPT)], dst_v)
    for g in range(ECH // 16):
        ones_v[pl.ds(g * 16, 16)] = jnp.ones((16,), jnp.float32)
    _drain_zero(s, stage_v, acc_sh, sem_z)
    plsc.subcore_barrier()

    def chunk_q(ci, b):
        base = ci * ECH
        for g in range(ECH // 16):
            o = base + g * 16
            sv = src_v[pl.ds(o, 16)]
            dv = dst_v[pl.ds(o, 16)]
            ibuf[b, pl.ds(g * 16, 16)] = jnp.minimum(
                sv * 128 + (dv % NPER), DUMP)

    _run_ring(chunk_q, NECH, ibuf, ones_v, acc_sh, sem_s)
    plsc.subcore_barrier()
    _copy_out(c, s, stage_v, acc_sh, q_out)


def _sc_mesh_kwargs():
    return dict(
        mesh=plsc.VectorSubcoreMesh(
            core_axis_name="c", subcore_axis_name="s",
            num_cores=NC, num_subcores=NS),
        compiler_params=pltpu.CompilerParams(needs_layout_passes=False),
    )


_COMMON_SCRATCH = [
    pltpu.VMEM((EPT,), jnp.int32),
    pltpu.VMEM((EPT,), jnp.int32),
    pltpu.VMEM((NB, ECH), jnp.int32),
    pltpu.VMEM((ECH,), jnp.float32),
    pltpu.VMEM((STG,), jnp.float32),
    pltpu.VMEM_SHARED((ACC_SZ,), jnp.float32),
    pltpu.SemaphoreType.DMA,
    pltpu.SemaphoreType.DMA,
]


@functools.cache
def _sc_cnt_fn():
    return pl.kernel(
        _sc_cnt_body,
        out_type=jax.ShapeDtypeStruct((NC * ACC_SZ,), jnp.float32),
        scratch_types=[pltpu.VMEM((NPAD // 4,), jnp.int32)] + _COMMON_SCRATCH,
        **_sc_mesh_kwargs(),
    )


@functools.cache
def _sc_q_fn():
    return pl.kernel(
        _sc_q_body,
        out_type=jax.ShapeDtypeStruct((NC * ACC_SZ,), jnp.float32),
        scratch_types=list(_COMMON_SCRATCH),
        **_sc_mesh_kwargs(),
    )


RB = 1000
NRB = N // RB


def _t1_body(cnt_ref, emb_ref, w1_ref, b1_ref, st_ref, p1_scr, acc):
    i = pl.program_id(0)

    @pl.when(i == 0)
    def _():
        p1_scr[...] = jnp.zeros_like(p1_scr)
        p1_scr[0:V, :] = jnp.dot(emb_ref[...], w1_ref[...],
                                 preferred_element_type=jnp.float32)
        acc[...] = jnp.zeros_like(acc)

    m = cnt_ref[0] + cnt_ref[1]
    t_blk = jnp.dot(m, p1_scr[...],
                    preferred_element_type=jnp.float32) + b1_ref[...]
    acc[0:1, :] += jnp.sum(t_blk, axis=0, keepdims=True)
    acc[1:2, :] += jnp.sum(t_blk * t_blk, axis=0, keepdims=True)
    st_ref[...] = acc[...]


_t1_in_specs = [
    pl.BlockSpec((NC, RB, 128), lambda i: (0, i, 0)),
    pl.BlockSpec((V, D), lambda i: (0, 0)),
    pl.BlockSpec((D, H), lambda i: (0, 0)),
    pl.BlockSpec((1, H), lambda i: (0, 0)),
]
_t1_out_specs = [
    pl.BlockSpec((2, H), lambda i: (0, 0)),
]
_t1_out_shape = [
    jax.ShapeDtypeStruct((2, H), jnp.float32),
]
_t1_scratch = [
    pltpu.VMEM((128, H), jnp.float32),
    pltpu.VMEM((2, H), jnp.float32),
]

_t1 = pl.pallas_call(
    _t1_body,
    grid=(NRB,),
    in_specs=_t1_in_specs,
    out_specs=_t1_out_specs,
    out_shape=_t1_out_shape,
    scratch_shapes=_t1_scratch,
)

GPB = RB // NPER


def _t2_body(cnt_ref, st_ref, emb_ref, w1_ref, b1_ref, g1_ref, be_ref,
             w2_ref, b2_ref, ws_ref, bs_ref, q_ref, out_ref, p1_scr, acc_sm):
    i = pl.program_id(0)

    @pl.when(i == 0)
    def _():
        p1_scr[...] = jnp.zeros_like(p1_scr)
        p1_scr[0:V, :] = jnp.dot(emb_ref[...], w1_ref[...],
                                 preferred_element_type=jnp.float32)
        acc_sm[0] = 0.0

    m = cnt_ref[0] + cnt_ref[1]
    t_blk = jnp.dot(m, p1_scr[...],
                    preferred_element_type=jnp.float32) + b1_ref[...]

    mu = st_ref[0:1, :] * (1.0 / N)
    var = st_ref[1:2, :] * (1.0 / N) - mu * mu
    inv = lax.rsqrt(var + 1e-5)
    y = (t_blk - mu) * inv * g1_ref[...] + be_ref[...]
    y = jnp.where(y > 0, y, jnp.exp(y) - 1.0)
    h2 = jnp.dot(y, w2_ref[...],
                 preferred_element_type=jnp.float32) + b2_ref[...]
    h2 = jnp.where(h2 > 0, h2, jnp.exp(h2) - 1.0)
    z = jnp.dot(h2, ws_ref[...],
                preferred_element_type=jnp.float32) + bs_ref[...]
    z = z - jnp.max(z, axis=-1, keepdims=True)
    ez = jnp.exp(z)
    sm = ez / jnp.sum(ez, axis=-1, keepdims=True)
    qs = q_ref[0] + q_ref[1]

    eye = (lax.broadcasted_iota(jnp.int32, (K, K), 0) ==
           lax.broadcasted_iota(jnp.int32, (K, K), 1)).astype(jnp.float32)
    zpad = jnp.zeros((128 - NPER, K), jnp.float32)

    tot = 0.0
    for g in range(GPB):
        sg = sm[g * NPER:(g + 1) * NPER, :]
        qb = qs[g * NPER:(g + 1) * NPER, :]
        deg = jnp.sum(qb, axis=1, keepdims=True)
        m2 = jnp.sum(deg)
        spad = jnp.concatenate([sg, zpad], axis=0)
        u = jnp.dot(qb, spad, preferred_element_type=jnp.float32)
        tr_out = jnp.sum(u * sg)
        cvec = jnp.sum(deg * sg, axis=0)
        tr_norm = jnp.sum(cvec * cvec) / m2
        spec = -(tr_out - tr_norm) / m2
        ss = lax.dot_general(sg, sg, (((0,), (0,)), ((), ())),
                             preferred_element_type=jnp.float32)
        fr = jnp.sqrt(jnp.sum(ss * ss))
        ortho = jnp.sqrt(jnp.sum((ss / fr - eye * 0.25) ** 2))
        cs = jnp.sum(sg, axis=0)
        clus = jnp.sqrt(jnp.sum(cs * cs)) * (4.0 / NPER) - 1.0
        tot = tot + spec + ortho + clus

    acc_sm[0] += tot
    out_ref[...] = jnp.broadcast_to(acc_sm[0] * (1.0 / B), (1, 1))


_t2_in_specs = [
    pl.BlockSpec((NC, RB, 128), lambda i: (0, i, 0)),
    pl.BlockSpec((2, H), lambda i: (0, 0)),
    pl.BlockSpec((V, D), lambda i: (0, 0)),
    pl.BlockSpec((D, H), lambda i: (0, 0)),
    pl.BlockSpec((1, H), lambda i: (0, 0)),
    pl.BlockSpec((1, H), lambda i: (0, 0)),
    pl.BlockSpec((1, H), lambda i: (0, 0)),
    pl.BlockSpec((H, H), lambda i: (0, 0)),
    pl.BlockSpec((1, H), lambda i: (0, 0)),
    pl.BlockSpec((H, K), lambda i: (0, 0)),
    pl.BlockSpec((1, K), lambda i: (0, 0)),
    pl.BlockSpec((NC, RB, 128), lambda i: (0, i, 0)),
]
_t2_out_specs = pl.BlockSpec((1, 1), lambda i: (0, 0))
_t2_out_shape = jax.ShapeDtypeStruct((1, 1), jnp.float32)
_t2_scratch = [
    pltpu.VMEM((128, H), jnp.float32),
    pltpu.SMEM((1,), jnp.float32),
]

_t2 = pl.pallas_call(
    _t2_body,
    grid=(NRB,),
    in_specs=_t2_in_specs,
    out_specs=_t2_out_specs,
    out_shape=_t2_out_shape,
    scratch_shapes=_t2_scratch,
)


def kernel(x, edge_index, batch, emb, W1, b1, gamma1, beta1, W2, b2, Ws, bs):
    x0 = x[:, 0].astype(jnp.int32)
    x0p = jnp.concatenate([x0, jnp.zeros((NPAD - N,), jnp.int32)])
    x0q = x0p.reshape(NPAD // 4, 4)
    x0w = (x0q[:, 0] | (x0q[:, 1] << 8) | (x0q[:, 2] << 16)
           | (x0q[:, 3] << 24))
    src = edge_index[0].astype(jnp.int32)
    dst = edge_index[1].astype(jnp.int32)
    epad = jnp.full((EPAD - E,), N, jnp.int32)
    srcp = jnp.concatenate([src, epad])
    dstp = jnp.concatenate([dst, epad])

    cnt_flat = _sc_cnt_fn()(srcp, dstp, x0w)
    q_flat = _sc_q_fn()(srcp, dstp)
    cnt3 = cnt_flat.reshape(NC, ROWS, 128)
    q3 = q_flat.reshape(NC, ROWS, 128)

    b1r = b1.reshape(1, H)
    (stats,) = _t1(cnt3, emb, W1, b1r)
    out = _t2(cnt3, stats, emb, W1, b1r, gamma1.reshape(1, H),
              beta1.reshape(1, H), W2, b2.reshape(1, H), Ws,
              bs.reshape(1, K), q3)
    return out.reshape(())


# speedup vs baseline: 38.2762x; 1.2756x over previous
import functools

import numpy as np

import jax
import jax.numpy as jnp
from jax import lax
from jax.experimental import pallas as pl
from jax.experimental.pallas import tpu as pltpu
from jax.experimental.pallas import tpu_sc as plsc

N = 10000
E = 160000
V = 64
D = 256
H = 512
K = 16
B = 100
NPER = 100

NC = 2
NS = 16
NW = NC * NS

ROWS = 10016
ACC_SZ = ROWS * 128
DUMP = N * 128

EPT = 5120
EPAD = EPT * NW
ECH = 128
NECH = EPT // ECH
NB = 4

NPT = 384
NPAD = NPT * NW
NNCH = NPT // ECH

SLC = ACC_SZ // NS
STG = 8192


def _chunks(total):
    offs = []
    o = 0
    while o < total:
        offs.append((o, min(STG, total - o)))
        o += STG
    return offs


def _zero_acc(s, stage_v, acc_sh, sem_z):
    def zfill(i, _):
        stage_v[pl.ds(i * 16, 16)] = jnp.zeros((16,), jnp.float32)
        return ()

    lax.fori_loop(0, STG // 16, zfill, ())
    for off, ln in _chunks(SLC):
        pltpu.async_copy(stage_v.at[pl.ds(0, ln)],
                         acc_sh.at[pl.ds(s * SLC + off, ln)], sem_z)


def _drain_zero(s, stage_v, acc_sh, sem_z):
    for off, ln in _chunks(SLC):
        pltpu.make_async_copy(stage_v.at[pl.ds(0, ln)],
                              acc_sh.at[pl.ds(s * SLC + off, ln)],
                              sem_z).wait()


def _copy_out(c, s, stage_v, acc_sh, out_hbm):
    for off, ln in _chunks(SLC):
        pltpu.sync_copy(acc_sh.at[pl.ds(s * SLC + off, ln)],
                        stage_v.at[pl.ds(0, ln)])
        pltpu.sync_copy(stage_v.at[pl.ds(0, ln)],
                        out_hbm.at[pl.ds(c * ACC_SZ + s * SLC + off, ln)])


def _run_ring(chunk_fn, nch, ibuf, ones_v, acc_sh, sem_s):
    def fire(b):
        pltpu.async_copy(ones_v, acc_sh.at[ibuf.at[b]], sem_s, add=True)

    def drain():
        pltpu.make_async_copy(ones_v, acc_sh.at[ibuf.at[0]], sem_s).wait()

    for b in range(NB):
        chunk_fn(b, b)
        fire(b)

    def ring_body(g4, _):
        for b in range(NB):
            drain()
            chunk_fn(NB + g4 * NB + b, b)
            fire(b)
        return ()

    lax.fori_loop(0, (nch - NB) // NB, ring_body, ())
    for b in range(NB):
        drain()


def _sc_cnt_body(src_hbm, dst_hbm, x0w_hbm, cnt_out,
                 x0w_v, src_v, dst_v, ibuf, ones_v, stage_v, acc_sh,
                 sem_s, sem_z):
    c = lax.axis_index("c")
    s = lax.axis_index("s")
    wid = s * NC + c

    _zero_acc(s, stage_v, acc_sh, sem_z)
    pltpu.sync_copy(src_hbm.at[pl.ds(wid * EPT, EPT)], src_v)
    pltpu.sync_copy(dst_hbm.at[pl.ds(wid * EPT, EPT)], dst_v)
    pltpu.sync_copy(x0w_hbm, x0w_v)
    for g in range(ECH // 16):
        ones_v[pl.ds(g * 16, 16)] = jnp.ones((16,), jnp.float32)
    _drain_zero(s, stage_v, acc_sh, sem_z)
    plsc.subcore_barrier()

    def lookup_x0(iv):
        wv = plsc.load_gather(x0w_v, [lax.shift_right_logical(iv, 2)])
        sh = (iv & 3) * 8
        return lax.shift_right_logical(wv, sh) & 63

    def chunk_cnt(ci, b):
        base = ci * ECH
        for g in range(ECH // 16):
            o = base + g * 16
            sv = src_v[pl.ds(o, 16)]
            dv = dst_v[pl.ds(o, 16)]
            xv = lookup_x0(sv)
            ibuf[b, pl.ds(g * 16, 16)] = jnp.minimum(dv * 128 + xv, DUMP)

    _run_ring(chunk_cnt, NECH, ibuf, ones_v, acc_sh, sem_s)

    def node_chunk(ci, _):
        base = wid * NPT + ci * ECH
        for g in range(ECH // 16):
            nv = base + g * 16 + lax.iota(jnp.int32, 16)
            xv = lookup_x0(nv)
            ibuf[0, pl.ds(g * 16, 16)] = jnp.minimum(nv * 128 + xv, DUMP)
        pltpu.sync_copy(ones_v, acc_sh.at[ibuf.at[0]], add=True)
        return ()

    lax.fori_loop(0, NNCH, node_chunk, ())
    plsc.subcore_barrier()
    _copy_out(c, s, stage_v, acc_sh, cnt_out)


def _sc_q_body(src_hbm, dst_hbm, q_out,
               src_v, dst_v, ibuf, ones_v, stage_v, acc_sh, sem_s, sem_z):
    c = lax.axis_index("c")
    s = lax.axis_index("s")
    wid = s * NC + c

    _zero_acc(s, stage_v, acc_sh, sem_z)
    pltpu.sync_copy(src_hbm.at[pl.ds(wid * EPT, EPT)], src_v)
    pltpu.sync_copy(dst_hbm.at[pl.ds(wid * EPT, EPT)], dst_v)
    for g in range(ECH // 16):
        ones_v[pl.ds(g * 16, 16)] = jnp.ones((16,), jnp.float32)
    _drain_zero(s, stage_v, acc_sh, sem_z)
    plsc.subcore_barrier()

    crec = float(np.nextafter(np.float32(0.01), np.float32(1)))

    def chunk_q(ci, b):
        base = ci * ECH
        for g in range(ECH // 16):
            o = base + g * 16
            sv = src_v[pl.ds(o, 16)]
            dv = dst_v[pl.ds(o, 16)]
            gq = (dv.astype(jnp.float32) * crec).astype(jnp.int32)
            ibuf[b, pl.ds(g * 16, 16)] = jnp.minimum(
                sv * 128 + (dv - gq * NPER), DUMP)

    _run_ring(chunk_q, NECH, ibuf, ones_v, acc_sh, sem_s)
    plsc.subcore_barrier()
    _copy_out(c, s, stage_v, acc_sh, q_out)


def _sc_mesh_kwargs():
    return dict(
        mesh=plsc.VectorSubcoreMesh(
            core_axis_name="c", subcore_axis_name="s",
            num_cores=NC, num_subcores=NS),
        compiler_params=pltpu.CompilerParams(needs_layout_passes=False),
    )


_COMMON_SCRATCH = [
    pltpu.VMEM((EPT,), jnp.int32),
    pltpu.VMEM((EPT,), jnp.int32),
    pltpu.VMEM((NB, ECH), jnp.int32),
    pltpu.VMEM((ECH,), jnp.float32),
    pltpu.VMEM((STG,), jnp.float32),
    pltpu.VMEM_SHARED((ACC_SZ,), jnp.float32),
    pltpu.SemaphoreType.DMA,
    pltpu.SemaphoreType.DMA,
]


@functools.cache
def _sc_cnt_fn():
    return pl.kernel(
        _sc_cnt_body,
        out_type=jax.ShapeDtypeStruct((NC * ACC_SZ,), jnp.float32),
        scratch_types=[pltpu.VMEM((NPAD // 4,), jnp.int32)] + _COMMON_SCRATCH,
        **_sc_mesh_kwargs(),
    )


@functools.cache
def _sc_q_fn():
    return pl.kernel(
        _sc_q_body,
        out_type=jax.ShapeDtypeStruct((NC * ACC_SZ,), jnp.float32),
        scratch_types=list(_COMMON_SCRATCH),
        **_sc_mesh_kwargs(),
    )


RB = 1000
NRB = N // RB


def _t1_body(cnt_ref, emb_ref, w1_ref, b1_ref, st_ref, p1_scr, acc):
    i = pl.program_id(0)

    @pl.when(i == 0)
    def _():
        p1_scr[...] = jnp.zeros_like(p1_scr)
        p1_scr[0:V, :] = jnp.dot(emb_ref[...], w1_ref[...],
                                 preferred_element_type=jnp.float32)
        acc[...] = jnp.zeros_like(acc)

    m = cnt_ref[0] + cnt_ref[1]
    t_blk = jnp.dot(m, p1_scr[...],
                    preferred_element_type=jnp.float32) + b1_ref[...]
    acc[0:1, :] += jnp.sum(t_blk, axis=0, keepdims=True)
    acc[1:2, :] += jnp.sum(t_blk * t_blk, axis=0, keepdims=True)
    st_ref[...] = acc[...]


_t1_in_specs = [
    pl.BlockSpec((NC, RB, 128), lambda i: (0, i, 0)),
    pl.BlockSpec((V, D), lambda i: (0, 0)),
    pl.BlockSpec((D, H), lambda i: (0, 0)),
    pl.BlockSpec((1, H), lambda i: (0, 0)),
]
_t1_out_specs = [
    pl.BlockSpec((2, H), lambda i: (0, 0)),
]
_t1_out_shape = [
    jax.ShapeDtypeStruct((2, H), jnp.float32),
]
_t1_scratch = [
    pltpu.VMEM((128, H), jnp.float32),
    pltpu.VMEM((2, H), jnp.float32),
]

_t1 = pl.pallas_call(
    _t1_body,
    grid=(NRB,),
    in_specs=_t1_in_specs,
    out_specs=_t1_out_specs,
    out_shape=_t1_out_shape,
    scratch_shapes=_t1_scratch,
)

GPB = RB // NPER


def _t2_body(cnt_ref, st_ref, emb_ref, w1_ref, b1_ref, g1_ref, be_ref,
             w2_ref, b2_ref, ws_ref, bs_ref, q_ref, out_ref, p1_scr, acc_sm):
    i = pl.program_id(0)

    @pl.when(i == 0)
    def _():
        p1_scr[...] = jnp.zeros_like(p1_scr)
        p1_scr[0:V, :] = jnp.dot(emb_ref[...], w1_ref[...],
                                 preferred_element_type=jnp.float32)
        acc_sm[0] = 0.0

    m = cnt_ref[0] + cnt_ref[1]
    t_blk = jnp.dot(m, p1_scr[...],
                    preferred_element_type=jnp.float32) + b1_ref[...]

    mu = st_ref[0:1, :] * (1.0 / N)
    var = st_ref[1:2, :] * (1.0 / N) - mu * mu
    inv = lax.rsqrt(var + 1e-5)
    y = (t_blk - mu) * inv * g1_ref[...] + be_ref[...]
    y = jnp.where(y > 0, y, jnp.exp(y) - 1.0)
    h2 = jnp.dot(y.astype(jnp.bfloat16), w2_ref[...].astype(jnp.bfloat16),
                 preferred_element_type=jnp.float32) + b2_ref[...]
    h2 = jnp.where(h2 > 0, h2, jnp.exp(h2) - 1.0)
    z = jnp.dot(h2, ws_ref[...],
                preferred_element_type=jnp.float32) + bs_ref[...]
    z = z - jnp.max(z, axis=-1, keepdims=True)
    ez = jnp.exp(z)
    sm = ez / jnp.sum(ez, axis=-1, keepdims=True)
    qs = q_ref[0] + q_ref[1]

    eye = (lax.broadcasted_iota(jnp.int32, (K, K), 0) ==
           lax.broadcasted_iota(jnp.int32, (K, K), 1)).astype(jnp.float32)
    zpad = jnp.zeros((128 - NPER, K), jnp.float32)

    tot = 0.0
    for g in range(GPB):
        sg = sm[g * NPER:(g + 1) * NPER, :]
        qb = qs[g * NPER:(g + 1) * NPER, :]
        deg = jnp.sum(qb, axis=1, keepdims=True)
        m2 = jnp.sum(deg)
        spad = jnp.concatenate([sg, zpad], axis=0)
        u = jnp.dot(qb, spad, preferred_element_type=jnp.float32)
        tr_out = jnp.sum(u * sg)
        cvec = jnp.sum(deg * sg, axis=0)
        tr_norm = jnp.sum(cvec * cvec) / m2
        spec = -(tr_out - tr_norm) / m2
        ss = lax.dot_general(sg, sg, (((0,), (0,)), ((), ())),
                             preferred_element_type=jnp.float32)
        fr = jnp.sqrt(jnp.sum(ss * ss))
        ortho = jnp.sqrt(jnp.sum((ss / fr - eye * 0.25) ** 2))
        cs = jnp.sum(sg, axis=0)
        clus = jnp.sqrt(jnp.sum(cs * cs)) * (4.0 / NPER) - 1.0
        tot = tot + spec + ortho + clus

    acc_sm[0] += tot
    out_ref[...] = jnp.broadcast_to(acc_sm[0] * (1.0 / B), (1, 1))


_t2_in_specs = [
    pl.BlockSpec((NC, RB, 128), lambda i: (0, i, 0)),
    pl.BlockSpec((2, H), lambda i: (0, 0)),
    pl.BlockSpec((V, D), lambda i: (0, 0)),
    pl.BlockSpec((D, H), lambda i: (0, 0)),
    pl.BlockSpec((1, H), lambda i: (0, 0)),
    pl.BlockSpec((1, H), lambda i: (0, 0)),
    pl.BlockSpec((1, H), lambda i: (0, 0)),
    pl.BlockSpec((H, H), lambda i: (0, 0)),
    pl.BlockSpec((1, H), lambda i: (0, 0)),
    pl.BlockSpec((H, K), lambda i: (0, 0)),
    pl.BlockSpec((1, K), lambda i: (0, 0)),
    pl.BlockSpec((NC, RB, 128), lambda i: (0, i, 0)),
]
_t2_out_specs = pl.BlockSpec((1, 1), lambda i: (0, 0))
_t2_out_shape = jax.ShapeDtypeStruct((1, 1), jnp.float32)
_t2_scratch = [
    pltpu.VMEM((128, H), jnp.float32),
    pltpu.SMEM((1,), jnp.float32),
]

_t2 = pl.pallas_call(
    _t2_body,
    grid=(NRB,),
    in_specs=_t2_in_specs,
    out_specs=_t2_out_specs,
    out_shape=_t2_out_shape,
    scratch_shapes=_t2_scratch,
)


def kernel(x, edge_index, batch, emb, W1, b1, gamma1, beta1, W2, b2, Ws, bs):
    x0 = x[:, 0].astype(jnp.int32)
    x0p = jnp.concatenate([x0, jnp.zeros((NPAD - N,), jnp.int32)])
    x0q = x0p.reshape(NPAD // 4, 4)
    x0w = (x0q[:, 0] | (x0q[:, 1] << 8) | (x0q[:, 2] << 16)
           | (x0q[:, 3] << 24))
    src = edge_index[0].astype(jnp.int32)
    dst = edge_index[1].astype(jnp.int32)
    epad = jnp.full((EPAD - E,), N, jnp.int32)
    srcp = jnp.concatenate([src, epad])
    dstp = jnp.concatenate([dst, epad])

    cnt_flat = _sc_cnt_fn()(srcp, dstp, x0w)
    q_flat = _sc_q_fn()(srcp, dstp)
    cnt3 = cnt_flat.reshape(NC, ROWS, 128)
    q3 = q_flat.reshape(NC, ROWS, 128)

    b1r = b1.reshape(1, H)
    (stats,) = _t1(cnt3, emb, W1, b1r)
    out = _t2(cnt3, stats, emb, W1, b1r, gamma1.reshape(1, H),
              beta1.reshape(1, H), W2, b2.reshape(1, H), Ws,
              bs.reshape(1, K), q3)
    return out.reshape(())


# speedup vs baseline: 42.0726x; 1.0992x over previous
import functools

import numpy as np

import jax
import jax.numpy as jnp
from jax import lax
from jax.experimental import pallas as pl
from jax.experimental.pallas import tpu as pltpu
from jax.experimental.pallas import tpu_sc as plsc

N = 10000
E = 160000
V = 64
D = 256
H = 512
K = 16
B = 100
NPER = 100

NC = 2
NS = 16
NW = NC * NS

ROWS = 10016
ACC_SZ = ROWS * 128
DUMP = N * 128

EPT = 5120
EPAD = EPT * NW
ECH = 128
NECH = EPT // ECH
NB = 4

NPT = 384
NPAD = NPT * NW
NNCH = NPT // ECH

SLC = ACC_SZ // NS
STG = 8192


def _chunks(total):
    offs = []
    o = 0
    while o < total:
        offs.append((o, min(STG, total - o)))
        o += STG
    return offs


def _zero_acc(s, stage_v, acc_sh, sem_z):
    def zfill(i, _):
        stage_v[pl.ds(i * 16, 16)] = jnp.zeros((16,), jnp.float32)
        return ()

    lax.fori_loop(0, STG // 16, zfill, ())
    for off, ln in _chunks(SLC):
        pltpu.async_copy(stage_v.at[pl.ds(0, ln)],
                         acc_sh.at[pl.ds(s * SLC + off, ln)], sem_z)


def _drain_zero(s, stage_v, acc_sh, sem_z):
    for off, ln in _chunks(SLC):
        pltpu.make_async_copy(stage_v.at[pl.ds(0, ln)],
                              acc_sh.at[pl.ds(s * SLC + off, ln)],
                              sem_z).wait()


def _copy_out(c, s, stage_v, acc_sh, out_hbm):
    for off, ln in _chunks(SLC):
        pltpu.sync_copy(acc_sh.at[pl.ds(s * SLC + off, ln)],
                        stage_v.at[pl.ds(0, ln)])
        pltpu.sync_copy(stage_v.at[pl.ds(0, ln)],
                        out_hbm.at[pl.ds(c * ACC_SZ + s * SLC + off, ln)])


def _run_ring(chunk_fn, nch, ibuf, ones_v, acc_sh, sem_s):
    def fire(b):
        pltpu.async_copy(ones_v, acc_sh.at[ibuf.at[b]], sem_s, add=True)

    def drain():
        pltpu.make_async_copy(ones_v, acc_sh.at[ibuf.at[0]], sem_s).wait()

    for b in range(NB):
        chunk_fn(b, b)
        fire(b)

    def ring_body(g4, _):
        for b in range(NB):
            drain()
            chunk_fn(NB + g4 * NB + b, b)
            fire(b)
        return ()

    lax.fori_loop(0, (nch - NB) // NB, ring_body, ())
    for b in range(NB):
        drain()


def _sc_cnt_body(src_hbm, dst_hbm, x0_hbm, cnt_out,
                 x0_v, src_v, dst_v, ibuf, ones_v, stage_v, acc_sh,
                 sem_s, sem_z):
    c = lax.axis_index("c")
    s = lax.axis_index("s")
    wid = s * NC + c

    _zero_acc(s, stage_v, acc_sh, sem_z)
    pltpu.sync_copy(src_hbm.at[pl.ds(wid * EPT, EPT)], src_v)
    pltpu.sync_copy(dst_hbm.at[pl.ds(wid * EPT, EPT)], dst_v)
    pltpu.sync_copy(x0_hbm, x0_v)
    for g in range(ECH // 16):
        ones_v[pl.ds(g * 16, 16)] = jnp.ones((16,), jnp.float32)
    _drain_zero(s, stage_v, acc_sh, sem_z)
    plsc.subcore_barrier()

    def lookup_x0(iv):
        return plsc.load_gather(x0_v, [iv])

    def chunk_cnt(ci, b):
        base = ci * ECH
        for g in range(ECH // 16):
            o = base + g * 16
            sv = src_v[pl.ds(o, 16)]
            dv = dst_v[pl.ds(o, 16)]
            xv = lookup_x0(sv)
            ibuf[b, pl.ds(g * 16, 16)] = jnp.minimum(dv * 128 + xv, DUMP)

    _run_ring(chunk_cnt, NECH, ibuf, ones_v, acc_sh, sem_s)

    def node_chunk(ci, _):
        base = wid * NPT + ci * ECH
        for g in range(ECH // 16):
            nv = base + g * 16 + lax.iota(jnp.int32, 16)
            xv = lookup_x0(nv)
            ibuf[0, pl.ds(g * 16, 16)] = jnp.minimum(nv * 128 + xv, DUMP)
        pltpu.sync_copy(ones_v, acc_sh.at[ibuf.at[0]], add=True)
        return ()

    lax.fori_loop(0, NNCH, node_chunk, ())
    plsc.subcore_barrier()
    _copy_out(c, s, stage_v, acc_sh, cnt_out)


def _sc_q_body(src_hbm, dst_hbm, q_out,
               src_v, dst_v, ibuf, ones_v, stage_v, acc_sh, sem_s, sem_z):
    c = lax.axis_index("c")
    s = lax.axis_index("s")
    wid = s * NC + c

    _zero_acc(s, stage_v, acc_sh, sem_z)
    pltpu.sync_copy(src_hbm.at[pl.ds(wid * EPT, EPT)], src_v)
    pltpu.sync_copy(dst_hbm.at[pl.ds(wid * EPT, EPT)], dst_v)
    for g in range(ECH // 16):
        ones_v[pl.ds(g * 16, 16)] = jnp.ones((16,), jnp.float32)
    _drain_zero(s, stage_v, acc_sh, sem_z)
    plsc.subcore_barrier()

    crec = float(np.nextafter(np.float32(0.01), np.float32(1)))

    def chunk_q(ci, b):
        base = ci * ECH
        for g in range(ECH // 16):
            o = base + g * 16
            sv = src_v[pl.ds(o, 16)]
            dv = dst_v[pl.ds(o, 16)]
            gq = (dv.astype(jnp.float32) * crec).astype(jnp.int32)
            ibuf[b, pl.ds(g * 16, 16)] = jnp.minimum(
                sv * 128 + (dv - gq * NPER), DUMP)

    _run_ring(chunk_q, NECH, ibuf, ones_v, acc_sh, sem_s)
    plsc.subcore_barrier()
    _copy_out(c, s, stage_v, acc_sh, q_out)


def _sc_mesh_kwargs():
    return dict(
        mesh=plsc.VectorSubcoreMesh(
            core_axis_name="c", subcore_axis_name="s",
            num_cores=NC, num_subcores=NS),
        compiler_params=pltpu.CompilerParams(needs_layout_passes=False),
    )


_COMMON_SCRATCH = [
    pltpu.VMEM((EPT,), jnp.int32),
    pltpu.VMEM((EPT,), jnp.int32),
    pltpu.VMEM((NB, ECH), jnp.int32),
    pltpu.VMEM((ECH,), jnp.float32),
    pltpu.VMEM((STG,), jnp.float32),
    pltpu.VMEM_SHARED((ACC_SZ,), jnp.float32),
    pltpu.SemaphoreType.DMA,
    pltpu.SemaphoreType.DMA,
]


@functools.cache
def _sc_cnt_fn():
    return pl.kernel(
        _sc_cnt_body,
        out_type=jax.ShapeDtypeStruct((NC * ACC_SZ,), jnp.float32),
        scratch_types=[pltpu.VMEM((NPAD,), jnp.int32)] + _COMMON_SCRATCH,
        **_sc_mesh_kwargs(),
    )


@functools.cache
def _sc_q_fn():
    return pl.kernel(
        _sc_q_body,
        out_type=jax.ShapeDtypeStruct((NC * ACC_SZ,), jnp.float32),
        scratch_types=list(_COMMON_SCRATCH),
        **_sc_mesh_kwargs(),
    )


RB = 1000
NRB = N // RB


def _t1_body(cnt_ref, emb_ref, w1_ref, b1_ref, st_ref, p1_scr, acc):
    i = pl.program_id(0)

    @pl.when(i == 0)
    def _():
        p1_scr[...] = jnp.zeros_like(p1_scr)
        p1_scr[0:V, :] = jnp.dot(emb_ref[...], w1_ref[...],
                                 preferred_element_type=jnp.float32)
        acc[...] = jnp.zeros_like(acc)

    m = cnt_ref[0] + cnt_ref[1]
    t_blk = jnp.dot(m, p1_scr[...],
                    preferred_element_type=jnp.float32) + b1_ref[...]
    acc[0:1, :] += jnp.sum(t_blk, axis=0, keepdims=True)
    acc[1:2, :] += jnp.sum(t_blk * t_blk, axis=0, keepdims=True)
    st_ref[...] = acc[...]


_t1_in_specs = [
    pl.BlockSpec((NC, RB, 128), lambda i: (0, i, 0)),
    pl.BlockSpec((V, D), lambda i: (0, 0)),
    pl.BlockSpec((D, H), lambda i: (0, 0)),
    pl.BlockSpec((1, H), lambda i: (0, 0)),
]
_t1_out_specs = [
    pl.BlockSpec((2, H), lambda i: (0, 0)),
]
_t1_out_shape = [
    jax.ShapeDtypeStruct((2, H), jnp.float32),
]
_t1_scratch = [
    pltpu.VMEM((128, H), jnp.float32),
    pltpu.VMEM((2, H), jnp.float32),
]

_t1 = pl.pallas_call(
    _t1_body,
    grid=(NRB,),
    in_specs=_t1_in_specs,
    out_specs=_t1_out_specs,
    out_shape=_t1_out_shape,
    scratch_shapes=_t1_scratch,
)

GPB = RB // NPER


def _t2a_body(cnt_ref, st_ref, emb_ref, w1_ref, b1_ref, g1_ref, be_ref,
              w2_ref, b2_ref, ws_ref, bs_ref, s_out, p1_scr):
    i = pl.program_id(0)

    @pl.when(i == 0)
    def _():
        p1_scr[...] = jnp.zeros_like(p1_scr)
        p1_scr[0:V, :] = jnp.dot(emb_ref[...], w1_ref[...],
                                 preferred_element_type=jnp.float32)

    m = cnt_ref[0] + cnt_ref[1]
    t_blk = jnp.dot(m, p1_scr[...],
                    preferred_element_type=jnp.float32) + b1_ref[...]

    mu = st_ref[0:1, :] * (1.0 / N)
    var = st_ref[1:2, :] * (1.0 / N) - mu * mu
    inv = lax.rsqrt(var + 1e-5)
    y = (t_blk - mu) * inv * g1_ref[...] + be_ref[...]
    y = jnp.where(y > 0, y, jnp.exp(y) - 1.0)
    h2 = jnp.dot(y.astype(jnp.bfloat16), w2_ref[...].astype(jnp.bfloat16),
                 preferred_element_type=jnp.float32) + b2_ref[...]
    h2 = jnp.where(h2 > 0, h2, jnp.exp(h2) - 1.0)
    z = jnp.dot(h2, ws_ref[...],
                preferred_element_type=jnp.float32) + bs_ref[...]
    z = z - jnp.max(z, axis=-1, keepdims=True)
    ez = jnp.exp(z)
    s_out[...] = ez / jnp.sum(ez, axis=-1, keepdims=True)


_t2a_in_specs = [
    pl.BlockSpec((NC, RB, 128), lambda i: (0, i, 0)),
    pl.BlockSpec((2, H), lambda i: (0, 0)),
    pl.BlockSpec((V, D), lambda i: (0, 0)),
    pl.BlockSpec((D, H), lambda i: (0, 0)),
    pl.BlockSpec((1, H), lambda i: (0, 0)),
    pl.BlockSpec((1, H), lambda i: (0, 0)),
    pl.BlockSpec((1, H), lambda i: (0, 0)),
    pl.BlockSpec((H, H), lambda i: (0, 0)),
    pl.BlockSpec((1, H), lambda i: (0, 0)),
    pl.BlockSpec((H, K), lambda i: (0, 0)),
    pl.BlockSpec((1, K), lambda i: (0, 0)),
]
_t2a_out_specs = pl.BlockSpec((RB, K), lambda i: (i, 0))
_t2a_out_shape = jax.ShapeDtypeStruct((N, K), jnp.float32)
_t2a_scratch = [pltpu.VMEM((128, H), jnp.float32)]

_t2a = pl.pallas_call(
    _t2a_body,
    grid=(NRB,),
    in_specs=_t2a_in_specs,
    out_specs=_t2a_out_specs,
    out_shape=_t2a_out_shape,
    scratch_shapes=_t2a_scratch,
)


def _t2b_body(s_ref, q_ref, out_ref, acc_sm):
    i = pl.program_id(0)

    @pl.when(i == 0)
    def _():
        acc_sm[0] = 0.0

    sm = s_ref[...]
    qs = q_ref[0] + q_ref[1]

    eye = (lax.broadcasted_iota(jnp.int32, (K, K), 0) ==
           lax.broadcasted_iota(jnp.int32, (K, K), 1)).astype(jnp.float32)
    zpad = jnp.zeros((128 - NPER, K), jnp.float32)

    tot = 0.0
    for g in range(GPB):
        sg = sm[g * NPER:(g + 1) * NPER, :]
        qb = qs[g * NPER:(g + 1) * NPER, :]
        deg = jnp.sum(qb, axis=1, keepdims=True)
        m2 = jnp.sum(deg)
        spad = jnp.concatenate([sg, zpad], axis=0)
        u = jnp.dot(qb, spad, preferred_element_type=jnp.float32)
        tr_out = jnp.sum(u * sg)
        cvec = jnp.sum(deg * sg, axis=0)
        tr_norm = jnp.sum(cvec * cvec) / m2
        spec = -(tr_out - tr_norm) / m2
        ss = lax.dot_general(sg, sg, (((0,), (0,)), ((), ())),
                             preferred_element_type=jnp.float32)
        fr = jnp.sqrt(jnp.sum(ss * ss))
        ortho = jnp.sqrt(jnp.sum((ss / fr - eye * 0.25) ** 2))
        cs = jnp.sum(sg, axis=0)
        clus = jnp.sqrt(jnp.sum(cs * cs)) * (4.0 / NPER) - 1.0
        tot = tot + spec + ortho + clus

    acc_sm[0] += tot
    out_ref[...] = jnp.broadcast_to(acc_sm[0] * (1.0 / B), (1, 1))


_t2b_in_specs = [
    pl.BlockSpec((RB, K), lambda i: (i, 0)),
    pl.BlockSpec((NC, RB, 128), lambda i: (0, i, 0)),
]
_t2b_out_specs = pl.BlockSpec((1, 1), lambda i: (0, 0))
_t2b_out_shape = jax.ShapeDtypeStruct((1, 1), jnp.float32)
_t2b_scratch = [pltpu.SMEM((1,), jnp.float32)]

_t2b = pl.pallas_call(
    _t2b_body,
    grid=(NRB,),
    in_specs=_t2b_in_specs,
    out_specs=_t2b_out_specs,
    out_shape=_t2b_out_shape,
    scratch_shapes=_t2b_scratch,
)


def kernel(x, edge_index, batch, emb, W1, b1, gamma1, beta1, W2, b2, Ws, bs):
    x0 = x[:, 0].astype(jnp.int32)
    x0p = jnp.concatenate([x0, jnp.zeros((NPAD - N,), jnp.int32)])
    src = edge_index[0].astype(jnp.int32)
    dst = edge_index[1].astype(jnp.int32)
    epad = jnp.full((EPAD - E,), N, jnp.int32)
    srcp = jnp.concatenate([src, epad])
    dstp = jnp.concatenate([dst, epad])

    cnt_flat = _sc_cnt_fn()(srcp, dstp, x0p)
    q_flat = _sc_q_fn()(srcp, dstp)
    cnt3 = cnt_flat.reshape(NC, ROWS, 128)
    q3 = q_flat.reshape(NC, ROWS, 128)

    b1r = b1.reshape(1, H)
    (stats,) = _t1(cnt3, emb, W1, b1r)
    s = _t2a(cnt3, stats, emb, W1, b1r, gamma1.reshape(1, H),
             beta1.reshape(1, H), W2, b2.reshape(1, H), Ws,
             bs.reshape(1, K))
    out = _t2b(s, q3)
    return out.reshape(())


# speedup vs baseline: 52.3971x; 1.2454x over previous
import functools

import numpy as np

import jax
import jax.numpy as jnp
from jax import lax
from jax.experimental import pallas as pl
from jax.experimental.pallas import tpu as pltpu
from jax.experimental.pallas import tpu_sc as plsc

N = 10000
E = 160000
V = 64
D = 256
H = 512
K = 16
B = 100
NPER = 100

NC = 2
NS = 16
NW = NC * NS

ROWS = 10016
ACC_SZ = ROWS * 128
DUMP = N * 128

EPT = 5120
EPAD = EPT * NW
ECH = 128
NECH = EPT // ECH
NB = 4

NPT = 384
NPAD = NPT * NW
NNCH = NPT // ECH

SLC = ACC_SZ // NS
STG = 8192


def _chunks(total):
    offs = []
    o = 0
    while o < total:
        offs.append((o, min(STG, total - o)))
        o += STG
    return offs


def _zero_acc(s, stage_v, acc_sh, sem_z):
    def zfill(i, _):
        stage_v[pl.ds(i * 16, 16)] = jnp.zeros((16,), jnp.float32)
        return ()

    lax.fori_loop(0, STG // 16, zfill, ())
    for off, ln in _chunks(SLC):
        pltpu.async_copy(stage_v.at[pl.ds(0, ln)],
                         acc_sh.at[pl.ds(s * SLC + off, ln)], sem_z)


def _drain_zero(s, stage_v, acc_sh, sem_z):
    for off, ln in _chunks(SLC):
        pltpu.make_async_copy(stage_v.at[pl.ds(0, ln)],
                              acc_sh.at[pl.ds(s * SLC + off, ln)],
                              sem_z).wait()


def _copy_out(c, s, stage_v, acc_sh, out_hbm):
    for off, ln in _chunks(SLC):
        pltpu.sync_copy(acc_sh.at[pl.ds(s * SLC + off, ln)],
                        stage_v.at[pl.ds(0, ln)])
        pltpu.sync_copy(stage_v.at[pl.ds(0, ln)],
                        out_hbm.at[pl.ds(c * ACC_SZ + s * SLC + off, ln)])


def _run_ring(chunk_fn, nch, ibuf, ones_v, acc_sh, sem_s):
    def fire(b):
        pltpu.async_copy(ones_v, acc_sh.at[ibuf.at[b]], sem_s, add=True)

    def drain():
        pltpu.make_async_copy(ones_v, acc_sh.at[ibuf.at[0]], sem_s).wait()

    for b in range(NB):
        chunk_fn(b, b)
        fire(b)

    def ring_body(g4, _):
        for b in range(NB):
            drain()
            chunk_fn(NB + g4 * NB + b, b)
            fire(b)
        return ()

    lax.fori_loop(0, (nch - NB) // NB, ring_body, ())
    for b in range(NB):
        drain()


def _sc_cnt_body(src_hbm, dst_hbm, x0_hbm, cnt_out,
                 x0_v, src_v, dst_v, ibuf, ones_v, stage_v, acc_sh,
                 sem_s, sem_z):
    c = lax.axis_index("c")
    s = lax.axis_index("s")
    wid = s * NC + c

    _zero_acc(s, stage_v, acc_sh, sem_z)
    pltpu.sync_copy(src_hbm.at[pl.ds(wid * EPT, EPT)], src_v)
    pltpu.sync_copy(dst_hbm.at[pl.ds(wid * EPT, EPT)], dst_v)
    pltpu.sync_copy(x0_hbm, x0_v)
    for g in range(ECH // 16):
        ones_v[pl.ds(g * 16, 16)] = jnp.ones((16,), jnp.float32)
    _drain_zero(s, stage_v, acc_sh, sem_z)
    plsc.subcore_barrier()

    def lookup_x0(iv):
        return plsc.load_gather(x0_v, [iv])

    def chunk_cnt(ci, b):
        base = ci * ECH
        for g in range(ECH // 16):
            o = base + g * 16
            sv = src_v[pl.ds(o, 16)]
            dv = dst_v[pl.ds(o, 16)]
            xv = lookup_x0(sv)
            ibuf[b, pl.ds(g * 16, 16)] = jnp.minimum(dv * 128 + xv, DUMP)

    _run_ring(chunk_cnt, NECH, ibuf, ones_v, acc_sh, sem_s)

    def node_chunk(ci, _):
        base = wid * NPT + ci * ECH
        for g in range(ECH // 16):
            nv = base + g * 16 + lax.iota(jnp.int32, 16)
            xv = lookup_x0(nv)
            ibuf[0, pl.ds(g * 16, 16)] = jnp.minimum(nv * 128 + xv, DUMP)
        pltpu.sync_copy(ones_v, acc_sh.at[ibuf.at[0]], add=True)
        return ()

    lax.fori_loop(0, NNCH, node_chunk, ())
    plsc.subcore_barrier()
    _copy_out(c, s, stage_v, acc_sh, cnt_out)


def _sc_q_body(src_hbm, dst_hbm, q_out,
               src_v, dst_v, ibuf, ones_v, stage_v, acc_sh, sem_s, sem_z):
    c = lax.axis_index("c")
    s = lax.axis_index("s")
    wid = s * NC + c

    _zero_acc(s, stage_v, acc_sh, sem_z)
    pltpu.sync_copy(src_hbm.at[pl.ds(wid * EPT, EPT)], src_v)
    pltpu.sync_copy(dst_hbm.at[pl.ds(wid * EPT, EPT)], dst_v)
    for g in range(ECH // 16):
        ones_v[pl.ds(g * 16, 16)] = jnp.ones((16,), jnp.float32)
    _drain_zero(s, stage_v, acc_sh, sem_z)
    plsc.subcore_barrier()

    crec = float(np.nextafter(np.float32(0.01), np.float32(1)))

    def chunk_q(ci, b):
        base = ci * ECH
        for g in range(ECH // 16):
            o = base + g * 16
            sv = src_v[pl.ds(o, 16)]
            dv = dst_v[pl.ds(o, 16)]
            gq = (dv.astype(jnp.float32) * crec).astype(jnp.int32)
            ibuf[b, pl.ds(g * 16, 16)] = jnp.minimum(
                sv * 128 + (dv - gq * NPER), DUMP)

    _run_ring(chunk_q, NECH, ibuf, ones_v, acc_sh, sem_s)
    plsc.subcore_barrier()
    _copy_out(c, s, stage_v, acc_sh, q_out)


def _sc_mesh_kwargs():
    return dict(
        mesh=plsc.VectorSubcoreMesh(
            core_axis_name="c", subcore_axis_name="s",
            num_cores=NC, num_subcores=NS),
        compiler_params=pltpu.CompilerParams(needs_layout_passes=False),
    )


_COMMON_SCRATCH = [
    pltpu.VMEM((EPT,), jnp.int32),
    pltpu.VMEM((EPT,), jnp.int32),
    pltpu.VMEM((NB, ECH), jnp.int32),
    pltpu.VMEM((ECH,), jnp.float32),
    pltpu.VMEM((STG,), jnp.float32),
    pltpu.VMEM_SHARED((ACC_SZ,), jnp.float32),
    pltpu.SemaphoreType.DMA,
    pltpu.SemaphoreType.DMA,
]


@functools.cache
def _sc_cnt_fn():
    return pl.kernel(
        _sc_cnt_body,
        out_type=jax.ShapeDtypeStruct((NC * ACC_SZ,), jnp.float32),
        scratch_types=[pltpu.VMEM((NPAD,), jnp.int32)] + _COMMON_SCRATCH,
        **_sc_mesh_kwargs(),
    )


@functools.cache
def _sc_q_fn():
    return pl.kernel(
        _sc_q_body,
        out_type=jax.ShapeDtypeStruct((NC * ACC_SZ,), jnp.float32),
        scratch_types=list(_COMMON_SCRATCH),
        **_sc_mesh_kwargs(),
    )


RB = 1000
NRB = N // RB


def _t1_body(cnt_ref, emb_ref, w1_ref, b1_ref, st_ref, p1_scr, acc):
    i = pl.program_id(0)

    @pl.when(i == 0)
    def _():
        p1_scr[...] = jnp.zeros_like(p1_scr)
        p1_scr[0:V, :] = jnp.dot(emb_ref[...], w1_ref[...],
                                 preferred_element_type=jnp.float32)
        acc[...] = jnp.zeros_like(acc)

    m = cnt_ref[0] + cnt_ref[1]
    t_blk = jnp.dot(m, p1_scr[...],
                    preferred_element_type=jnp.float32) + b1_ref[...]
    acc[0:1, :] += jnp.sum(t_blk, axis=0, keepdims=True)
    acc[1:2, :] += jnp.sum(t_blk * t_blk, axis=0, keepdims=True)
    st_ref[...] = acc[...]


_t1_in_specs = [
    pl.BlockSpec((NC, RB, 128), lambda i: (0, i, 0)),
    pl.BlockSpec((V, D), lambda i: (0, 0)),
    pl.BlockSpec((D, H), lambda i: (0, 0)),
    pl.BlockSpec((1, H), lambda i: (0, 0)),
]
_t1_out_specs = [
    pl.BlockSpec((2, H), lambda i: (0, 0)),
]
_t1_out_shape = [
    jax.ShapeDtypeStruct((2, H), jnp.float32),
]
_t1_scratch = [
    pltpu.VMEM((128, H), jnp.float32),
    pltpu.VMEM((2, H), jnp.float32),
]

_t1 = pl.pallas_call(
    _t1_body,
    grid=(NRB,),
    in_specs=_t1_in_specs,
    out_specs=_t1_out_specs,
    out_shape=_t1_out_shape,
    scratch_shapes=_t1_scratch,
)

GPB = RB // NPER


def _t2a_body(cnt_ref, st_ref, emb_ref, w1_ref, b1_ref, g1_ref, be_ref,
              w2_ref, b2_ref, ws_ref, bs_ref, s_out, p1_scr):
    i = pl.program_id(0)

    @pl.when(i == 0)
    def _():
        p1_scr[...] = jnp.zeros_like(p1_scr)
        p1_scr[0:V, :] = jnp.dot(emb_ref[...], w1_ref[...],
                                 preferred_element_type=jnp.float32)

    m = cnt_ref[0] + cnt_ref[1]
    t_blk = jnp.dot(m, p1_scr[...],
                    preferred_element_type=jnp.float32) + b1_ref[...]

    mu = st_ref[0:1, :] * (1.0 / N)
    var = st_ref[1:2, :] * (1.0 / N) - mu * mu
    inv = lax.rsqrt(var + 1e-5)
    y = (t_blk - mu) * inv * g1_ref[...] + be_ref[...]
    y = jnp.where(y > 0, y, jnp.exp(y) - 1.0)
    h2 = jnp.dot(y.astype(jnp.bfloat16), w2_ref[...].astype(jnp.bfloat16),
                 preferred_element_type=jnp.float32) + b2_ref[...]
    h2 = jnp.where(h2 > 0, h2, jnp.exp(h2) - 1.0)
    z = jnp.dot(h2, ws_ref[...],
                preferred_element_type=jnp.float32) + bs_ref[...]
    z = z - jnp.max(z, axis=-1, keepdims=True)
    ez = jnp.exp(z)
    s_out[...] = ez / jnp.sum(ez, axis=-1, keepdims=True)


_t2a_in_specs = [
    pl.BlockSpec((NC, RB, 128), lambda i: (0, i, 0)),
    pl.BlockSpec((2, H), lambda i: (0, 0)),
    pl.BlockSpec((V, D), lambda i: (0, 0)),
    pl.BlockSpec((D, H), lambda i: (0, 0)),
    pl.BlockSpec((1, H), lambda i: (0, 0)),
    pl.BlockSpec((1, H), lambda i: (0, 0)),
    pl.BlockSpec((1, H), lambda i: (0, 0)),
    pl.BlockSpec((H, H), lambda i: (0, 0)),
    pl.BlockSpec((1, H), lambda i: (0, 0)),
    pl.BlockSpec((H, K), lambda i: (0, 0)),
    pl.BlockSpec((1, K), lambda i: (0, 0)),
]
_t2a_out_specs = pl.BlockSpec((RB, K), lambda i: (i, 0))
_t2a_out_shape = jax.ShapeDtypeStruct((N, K), jnp.float32)
_t2a_scratch = [pltpu.VMEM((128, H), jnp.float32)]

_t2a = pl.pallas_call(
    _t2a_body,
    grid=(NRB,),
    in_specs=_t2a_in_specs,
    out_specs=_t2a_out_specs,
    out_shape=_t2a_out_shape,
    scratch_shapes=_t2a_scratch,
)


def _t2b_body(s_ref, q_ref, out_ref, acc_sm):
    i = pl.program_id(0)

    @pl.when(i == 0)
    def _():
        acc_sm[0] = 0.0

    sm = s_ref[...]
    qs = q_ref[0] + q_ref[1]

    row = lax.broadcasted_iota(jnp.int32, (GPB, RB), 1)
    gid = lax.broadcasted_iota(jnp.int32, (GPB, RB), 0)
    emat = ((row >= gid * NPER) & (row < (gid + 1) * NPER)
            ).astype(jnp.float32)
    zpad = jnp.zeros((128 - NPER, K), jnp.float32)

    deg = jnp.sum(qs, axis=1, keepdims=True)
    gmat = jnp.dot(emat, sm * deg,
                   preferred_element_type=jnp.float32)
    csum = jnp.dot(emat, sm,
                   preferred_element_type=jnp.float32)
    m2v = jnp.dot(emat, deg,
                  preferred_element_type=jnp.float32)
    trss = jnp.dot(emat, jnp.sum(sm * sm, axis=1, keepdims=True),
                   preferred_element_type=jnp.float32)

    ulist = []
    sslist = []
    for g in range(GPB):
        sg = sm[g * NPER:(g + 1) * NPER, :]
        qb = qs[g * NPER:(g + 1) * NPER, :]
        spad = jnp.concatenate([sg, zpad], axis=0)
        ulist.append(jnp.dot(qb, spad,
                             preferred_element_type=jnp.float32))
        sslist.append(lax.dot_general(sg, sg, (((0,), (0,)), ((), ())),
                                      preferred_element_type=jnp.float32))
    u_all = jnp.concatenate(ulist, axis=0)
    ss_all = jnp.concatenate(sslist, axis=0)

    trv = jnp.dot(emat, jnp.sum(u_all * sm, axis=1, keepdims=True),
                  preferred_element_type=jnp.float32)
    rowg = lax.broadcasted_iota(jnp.int32, (GPB, GPB * K), 1)
    gidg = lax.broadcasted_iota(jnp.int32, (GPB, GPB * K), 0)
    emat16 = ((rowg >= gidg * K) & (rowg < (gidg + 1) * K)
              ).astype(jnp.float32)
    fr2 = jnp.dot(emat16, jnp.sum(ss_all * ss_all, axis=1, keepdims=True),
                  preferred_element_type=jnp.float32)

    tr_norm = jnp.sum(gmat * gmat, axis=1, keepdims=True) / m2v
    spec_v = -(trv - tr_norm) / m2v
    ortho_v = jnp.sqrt(2.0 - trss / (2.0 * jnp.sqrt(fr2)))
    clus_v = (jnp.sqrt(jnp.sum(csum * csum, axis=1, keepdims=True))
              * (4.0 / NPER) - 1.0)
    tot = jnp.sum(spec_v + ortho_v + clus_v)

    acc_sm[0] += tot
    out_ref[...] = jnp.broadcast_to(acc_sm[0] * (1.0 / B), (1, 1))


_t2b_in_specs = [
    pl.BlockSpec((RB, K), lambda i: (i, 0)),
    pl.BlockSpec((NC, RB, 128), lambda i: (0, i, 0)),
]
_t2b_out_specs = pl.BlockSpec((1, 1), lambda i: (0, 0))
_t2b_out_shape = jax.ShapeDtypeStruct((1, 1), jnp.float32)
_t2b_scratch = [pltpu.SMEM((1,), jnp.float32)]

_t2b = pl.pallas_call(
    _t2b_body,
    grid=(NRB,),
    in_specs=_t2b_in_specs,
    out_specs=_t2b_out_specs,
    out_shape=_t2b_out_shape,
    scratch_shapes=_t2b_scratch,
)


def kernel(x, edge_index, batch, emb, W1, b1, gamma1, beta1, W2, b2, Ws, bs):
    x0 = x[:, 0].astype(jnp.int32)
    x0p = jnp.concatenate([x0, jnp.zeros((NPAD - N,), jnp.int32)])
    src = edge_index[0].astype(jnp.int32)
    dst = edge_index[1].astype(jnp.int32)
    epad = jnp.full((EPAD - E,), N, jnp.int32)
    srcp = jnp.concatenate([src, epad])
    dstp = jnp.concatenate([dst, epad])

    cnt_flat = _sc_cnt_fn()(srcp, dstp, x0p)
    q_flat = _sc_q_fn()(srcp, dstp)
    cnt3 = cnt_flat.reshape(NC, ROWS, 128)
    q3 = q_flat.reshape(NC, ROWS, 128)

    b1r = b1.reshape(1, H)
    (stats,) = _t1(cnt3, emb, W1, b1r)
    s = _t2a(cnt3, stats, emb, W1, b1r, gamma1.reshape(1, H),
             beta1.reshape(1, H), W2, b2.reshape(1, H), Ws,
             bs.reshape(1, K))
    out = _t2b(s, q3)
    return out.reshape(())
